# Initial kernel scaffold; baseline (speedup 1.0000x reference)
import jax
import jax.numpy as jnp
from jax.experimental import pallas as pl


def kernel(x, edge_index, Wp, bp, W1, W2):
    raise NotImplementedError("write your pallas kernel here")


# speedup vs baseline: 1.1821x; 1.1821x over previous
import functools

import jax
import jax.numpy as jnp
from jax import lax
from jax.experimental import pallas as pl
from jax.experimental.pallas import tpu as pltpu
from jax.experimental.pallas import tpu_sc as plsc

_N_PAD = 10240
_NW = 32
_ROWS_W = _N_PAD // _NW
_CHUNK = 2000
_PEND = 128
_FLUSH_AT = _PEND - 16
_L = 16


def _dot(a, b):
    return jnp.dot(a, b, preferred_element_type=jnp.float32,
                   precision=lax.Precision.HIGHEST)


def _pre_body(h_ref, wpt_ref, bp_ref, o_ref):
    o_ref[...] = jax.nn.sigmoid(_dot(h_ref[...], wpt_ref[...]) + bp_ref[0:1, :])


def _tc_pre(h, wpt, bp8):
    d = h.shape[1]
    rb = 1024
    return pl.pallas_call(
        _pre_body,
        grid=(_N_PAD // rb,),
        in_specs=[
            pl.BlockSpec((rb, d), lambda i: (i, 0)),
            pl.BlockSpec((d, d), lambda i: (0, 0)),
            pl.BlockSpec((8, d), lambda i: (0, 0)),
        ],
        out_specs=pl.BlockSpec((rb, d), lambda i: (i, 0)),
        out_shape=jax.ShapeDtypeStruct((_N_PAD, d), jnp.float32),
    )(h, wpt, bp8)


def _post_update(h, ha, wat, wbt):
    s = jax.nn.sigmoid(_dot(h, wat) + _dot(ha, wbt))
    nrm = jnp.sqrt(jnp.sum(s * s, axis=1, keepdims=True))
    return s / jnp.maximum(nrm, 1e-12)


def _post_body(h_ref, ha_ref, wat_ref, wbt_ref, o_ref):
    o_ref[...] = _post_update(h_ref[...], ha_ref[...], wat_ref[...], wbt_ref[...])


def _post_hp_body(h_ref, ha_ref, wat_ref, wbt_ref, wpt_ref, bp_ref, o_ref,
                  ohp_ref):
    hn = _post_update(h_ref[...], ha_ref[...], wat_ref[...], wbt_ref[...])
    o_ref[...] = hn
    ohp_ref[...] = jax.nn.sigmoid(_dot(hn, wpt_ref[...]) + bp_ref[0:1, :])


def _tc_post(h, ha, wat, wbt):
    d = h.shape[1]
    rb = 1024
    return pl.pallas_call(
        _post_body,
        grid=(_N_PAD // rb,),
        in_specs=[
            pl.BlockSpec((rb, d), lambda i: (i, 0)),
            pl.BlockSpec((rb, d), lambda i: (i, 0)),
            pl.BlockSpec((d, d), lambda i: (0, 0)),
            pl.BlockSpec((d, d), lambda i: (0, 0)),
        ],
        out_specs=pl.BlockSpec((rb, d), lambda i: (i, 0)),
        out_shape=jax.ShapeDtypeStruct((_N_PAD, d), jnp.float32),
    )(h, ha, wat, wbt)


def _tc_post_hp(h, ha, wat, wbt, wpt, bp8):
    d = h.shape[1]
    rb = 1024
    return pl.pallas_call(
        _post_hp_body,
        grid=(_N_PAD // rb,),
        in_specs=[
            pl.BlockSpec((rb, d), lambda i: (i, 0)),
            pl.BlockSpec((rb, d), lambda i: (i, 0)),
            pl.BlockSpec((d, d), lambda i: (0, 0)),
            pl.BlockSpec((d, d), lambda i: (0, 0)),
            pl.BlockSpec((d, d), lambda i: (0, 0)),
            pl.BlockSpec((8, d), lambda i: (0, 0)),
        ],
        out_specs=[
            pl.BlockSpec((rb, d), lambda i: (i, 0)),
            pl.BlockSpec((rb, d), lambda i: (i, 0)),
        ],
        out_shape=[
            jax.ShapeDtypeStruct((_N_PAD, d), jnp.float32),
            jax.ShapeDtypeStruct((_N_PAD, d), jnp.float32),
        ],
    )(h, ha, wat, wbt, wpt, bp8)


def _segmax(hp, src, dst):
    d = hp.shape[1]
    e_pad = src.shape[0]
    n_chunks = e_pad // _CHUNK
    mesh = plsc.VectorSubcoreMesh(core_axis_name="c", subcore_axis_name="s")

    @functools.partial(
        pl.kernel,
        mesh=mesh,
        compiler_params=pltpu.CompilerParams(needs_layout_passes=False),
        out_type=jax.ShapeDtypeStruct((_N_PAD, d), jnp.float32),
        scratch_types=[
            pltpu.VMEM((_ROWS_W, d), jnp.float32),
            pltpu.VMEM((_CHUNK,), jnp.int32),
            pltpu.VMEM((_CHUNK,), jnp.int32),
            pltpu.VMEM((_PEND,), jnp.int32),
            pltpu.VMEM((_PEND + _L,), jnp.int32),
            pltpu.VMEM((_PEND, d), jnp.float32),
            pltpu.SMEM((1,), jnp.int32),
            pltpu.SemaphoreType.DMA,
        ],
    )
    def seg_kernel(hp_hbm, src_hbm, dst_hbm, out_hbm, acc, src_buf, dst_buf,
                   psrc, pdst, rows, cnt_ref, sem):
        cid = lax.axis_index("c")
        sid = lax.axis_index("s")
        wid = cid * 16 + sid
        lo = wid * _ROWS_W
        hi = lo + _ROWS_W

        fzero = jnp.zeros((_L,), jnp.float32)
        izero = jnp.zeros((_L,), jnp.int32)

        @pl.loop(0, _ROWS_W)
        def _(r):
            for c in range(d // _L):
                acc[r, pl.ds(c * _L, _L)] = fzero

        for c in range(_PEND // _L):
            psrc[pl.ds(c * _L, _L)] = izero
        cnt_ref[0] = 0

        def flush():
            k = cnt_ref[0]
            pltpu.async_copy(hp_hbm.at[psrc], rows, sem).wait()

            def fold(j, carry):
                ld = pdst[pl.ds(j, _L)][0]
                for c in range(d // _L):
                    sl = pl.ds(c * _L, _L)
                    acc[ld, sl] = jnp.maximum(acc[ld, sl], rows[j, sl])
                return carry

            lax.fori_loop(0, k, fold, 0)
            cnt_ref[0] = 0

        @pl.loop(0, n_chunks)
        def _(cix):
            base = cix * _CHUNK
            pltpu.sync_copy(src_hbm.at[pl.ds(base, _CHUNK)], src_buf)
            pltpu.sync_copy(dst_hbm.at[pl.ds(base, _CHUNK)], dst_buf)

            @pl.loop(0, _CHUNK, step=_L)
            def _(i):
                dvec = dst_buf[pl.ds(i, _L)]
                svec = src_buf[pl.ds(i, _L)]
                m = (dvec >= lo) & (dvec < hi)
                k = cnt_ref[0]
                plsc.store_compressed(psrc.at[pl.ds(k, _L)], svec, mask=m)
                plsc.store_compressed(pdst.at[pl.ds(k, _L)], dvec - lo, mask=m)
                cnt_ref[0] = k + jnp.sum(m.astype(jnp.int32), axis=0)

                @pl.when(cnt_ref[0] > _FLUSH_AT)
                def _():
                    flush()

        @pl.when(cnt_ref[0] > 0)
        def _():
            flush()

        pltpu.sync_copy(acc, out_hbm.at[pl.ds(lo, _ROWS_W)])

    return seg_kernel(hp, src, dst)


def kernel(x, edge_index, Wp, bp, W1, W2):
    n, d = x.shape
    src = edge_index[0].astype(jnp.int32)
    dst = edge_index[1].astype(jnp.int32)
    e = src.shape[0]
    e_pad = ((e + _CHUNK - 1) // _CHUNK) * _CHUNK
    if e_pad != e:
        src = jnp.concatenate([src, jnp.zeros((e_pad - e,), jnp.int32)])
        dst = jnp.concatenate([dst, jnp.full((e_pad - e,), n, jnp.int32)])

    x_pad = jnp.zeros((_N_PAD, d), jnp.float32).at[:n].set(x)
    wpt = Wp.T
    bp8 = jnp.broadcast_to(bp.reshape(1, d), (8, d))
    w1t = W1.T
    w2t = W2.T
    wat1, wbt1 = w1t[:d], w1t[d:]
    wat2, wbt2 = w2t[:d], w2t[d:]

    hp1 = _tc_pre(x_pad, wpt, bp8)
    ha1 = _segmax(hp1, src, dst)
    h2, hp2 = _tc_post_hp(x_pad, ha1, wat1, wbt1, wpt, bp8)
    ha2 = _segmax(hp2, src, dst)
    h3 = _tc_post(h2, ha2, wat2, wbt2)
    return h3[:n]


# speedup vs baseline: 1.7391x; 1.4712x over previous
import functools

import jax
import jax.numpy as jnp
from jax import lax
from jax.experimental import pallas as pl
from jax.experimental.pallas import tpu as pltpu
from jax.experimental.pallas import tpu_sc as plsc

_N_PAD = 10240
_NW = 32
_ROWS_W = _N_PAD // _NW
_CHUNK = 2000
_BLK = 64
_L = 16

_SC_PARAMS = pltpu.CompilerParams(needs_layout_passes=False)


def _dot(a, b):
    return jnp.dot(a, b, preferred_element_type=jnp.float32,
                   precision=lax.Precision.HIGHEST)


def _pre_body(h_ref, wpt_ref, bp_ref, o_ref):
    o_ref[...] = jax.nn.sigmoid(_dot(h_ref[...], wpt_ref[...]) + bp_ref[0:1, :])


def _tc_pre(h, wpt, bp8):
    d = h.shape[1]
    rb = 1024
    return pl.pallas_call(
        _pre_body,
        grid=(_N_PAD // rb,),
        in_specs=[
            pl.BlockSpec((rb, d), lambda i: (i, 0)),
            pl.BlockSpec((d, d), lambda i: (0, 0)),
            pl.BlockSpec((8, d), lambda i: (0, 0)),
        ],
        out_specs=pl.BlockSpec((rb, d), lambda i: (i, 0)),
        out_shape=jax.ShapeDtypeStruct((_N_PAD, d), jnp.float32),
    )(h, wpt, bp8)


def _post_update(h, ha, wat, wbt):
    s = jax.nn.sigmoid(_dot(h, wat) + _dot(ha, wbt))
    nrm = jnp.sqrt(jnp.sum(s * s, axis=1, keepdims=True))
    return s / jnp.maximum(nrm, 1e-12)


def _post_body(h_ref, ha_ref, wat_ref, wbt_ref, o_ref):
    o_ref[...] = _post_update(h_ref[...], ha_ref[...], wat_ref[...], wbt_ref[...])


def _post_hp_body(h_ref, ha_ref, wat_ref, wbt_ref, wpt_ref, bp_ref, o_ref,
                  ohp_ref):
    hn = _post_update(h_ref[...], ha_ref[...], wat_ref[...], wbt_ref[...])
    o_ref[...] = hn
    ohp_ref[...] = jax.nn.sigmoid(_dot(hn, wpt_ref[...]) + bp_ref[0:1, :])


def _tc_post(h, ha, wat, wbt):
    d = h.shape[1]
    rb = 1024
    return pl.pallas_call(
        _post_body,
        grid=(_N_PAD // rb,),
        in_specs=[
            pl.BlockSpec((rb, d), lambda i: (i, 0)),
            pl.BlockSpec((rb, d), lambda i: (i, 0)),
            pl.BlockSpec((d, d), lambda i: (0, 0)),
            pl.BlockSpec((d, d), lambda i: (0, 0)),
        ],
        out_specs=pl.BlockSpec((rb, d), lambda i: (i, 0)),
        out_shape=jax.ShapeDtypeStruct((_N_PAD, d), jnp.float32),
    )(h, ha, wat, wbt)


def _tc_post_hp(h, ha, wat, wbt, wpt, bp8):
    d = h.shape[1]
    rb = 1024
    return pl.pallas_call(
        _post_hp_body,
        grid=(_N_PAD // rb,),
        in_specs=[
            pl.BlockSpec((rb, d), lambda i: (i, 0)),
            pl.BlockSpec((rb, d), lambda i: (i, 0)),
            pl.BlockSpec((d, d), lambda i: (0, 0)),
            pl.BlockSpec((d, d), lambda i: (0, 0)),
            pl.BlockSpec((d, d), lambda i: (0, 0)),
            pl.BlockSpec((8, d), lambda i: (0, 0)),
        ],
        out_specs=[
            pl.BlockSpec((rb, d), lambda i: (i, 0)),
            pl.BlockSpec((rb, d), lambda i: (i, 0)),
        ],
        out_shape=[
            jax.ShapeDtypeStruct((_N_PAD, d), jnp.float32),
            jax.ShapeDtypeStruct((_N_PAD, d), jnp.float32),
        ],
    )(h, ha, wat, wbt, wpt, bp8)


def _edge_partition(src, dst):
    e_pad = src.shape[0]
    n_chunks = e_pad // _CHUNK
    n_pairs = (n_chunks + 1) // 2
    cap = ((e_pad + _BLK - 1) // _BLK) * _BLK
    mesh = plsc.VectorSubcoreMesh(core_axis_name="c", subcore_axis_name="s")

    @functools.partial(
        pl.kernel,
        mesh=mesh,
        compiler_params=_SC_PARAMS,
        out_type=[
            jax.ShapeDtypeStruct((_NW, cap), jnp.int32),
            jax.ShapeDtypeStruct((_NW, cap), jnp.int32),
            jax.ShapeDtypeStruct((_NW, _L), jnp.int32),
        ],
        scratch_types=[
            pltpu.VMEM((_CHUNK,), jnp.int32),
            pltpu.VMEM((_CHUNK,), jnp.int32),
            pltpu.VMEM((_CHUNK,), jnp.int32),
            pltpu.VMEM((_CHUNK,), jnp.int32),
            pltpu.VMEM((_BLK + _L,), jnp.int32),
            pltpu.VMEM((_BLK + _L,), jnp.int32),
            pltpu.SMEM((2,), jnp.int32),
            pltpu.VMEM((_L,), jnp.int32),
            pltpu.SemaphoreType.DMA,
            pltpu.SemaphoreType.DMA,
        ],
    )
    def part_kernel(src_hbm, dst_hbm, lsrc_hbm, ldst_hbm, cnt_hbm,
                    src_a, src_b, dst_a, dst_b, psrc, pdst, st, crow,
                    sem0, sem1):
        cid = lax.axis_index("c")
        sid = lax.axis_index("s")
        wid = cid * 16 + sid
        lo = wid * _ROWS_W
        hi = lo + _ROWS_W

        izero = jnp.zeros((_L,), jnp.int32)
        for c in range((_BLK + _L) // _L):
            psrc[pl.ds(c * _L, _L)] = izero
        st[0] = 0
        st[1] = 0

        def stage(cix, sbuf, dbuf, sem):
            base = cix * _CHUNK
            pltpu.async_copy(src_hbm.at[pl.ds(base, _CHUNK)], sbuf, sem)
            pltpu.async_copy(dst_hbm.at[pl.ds(base, _CHUNK)], dbuf, sem)

        def wait(cix, sbuf, dbuf, sem):
            pltpu.make_async_copy(src_hbm.at[pl.ds(cix * _CHUNK, _CHUNK)],
                                  sbuf, sem).wait()
            pltpu.make_async_copy(dst_hbm.at[pl.ds(cix * _CHUNK, _CHUNK)],
                                  dbuf, sem).wait()

        def emit_block():
            b = st[1]
            pltpu.sync_copy(psrc.at[pl.ds(0, _BLK)],
                            lsrc_hbm.at[wid, pl.ds(b * _BLK, _BLK)])
            pltpu.sync_copy(pdst.at[pl.ds(0, _BLK)],
                            ldst_hbm.at[wid, pl.ds(b * _BLK, _BLK)])
            st[1] = b + 1

        def scan(sbuf, dbuf):
            @pl.loop(0, _CHUNK, step=_L)
            def _(i):
                dvec = dbuf[pl.ds(i, _L)]
                svec = sbuf[pl.ds(i, _L)]
                m = (dvec >= lo) & (dvec < hi)
                k = st[0]
                plsc.store_compressed(psrc.at[pl.ds(k, _L)], svec, mask=m)
                plsc.store_compressed(pdst.at[pl.ds(k, _L)], dvec - lo, mask=m)
                st[0] = k + jnp.sum(m.astype(jnp.int32), axis=0)

                @pl.when(st[0] >= _BLK)
                def _():
                    emit_block()
                    psrc[pl.ds(0, _L)] = psrc[pl.ds(_BLK, _L)]
                    pdst[pl.ds(0, _L)] = pdst[pl.ds(_BLK, _L)]
                    st[0] = st[0] - _BLK

        stage(0, src_a, dst_a, sem0)
        stage(1, src_b, dst_b, sem1)

        @pl.loop(0, n_pairs)
        def _(t):
            c0 = 2 * t
            c1 = 2 * t + 1
            wait(c0, src_a, dst_a, sem0)
            scan(src_a, dst_a)

            @pl.when(c0 + 2 < n_chunks)
            def _():
                stage(c0 + 2, src_a, dst_a, sem0)

            @pl.when(c1 < n_chunks)
            def _():
                wait(c1, src_b, dst_b, sem1)
                scan(src_b, dst_b)

                @pl.when(c1 + 2 < n_chunks)
                def _():
                    stage(c1 + 2, src_b, dst_b, sem1)

        total = st[1] * _BLK + st[0]

        @pl.when(st[0] > 0)
        def _():
            emit_block()

        crow[...] = jnp.full((_L,), total, jnp.int32)
        pltpu.sync_copy(crow, cnt_hbm.at[wid])

    return part_kernel(src, dst)


def _fold(hp, lsrc, ldst, cnt):
    d = hp.shape[1]
    mesh = plsc.VectorSubcoreMesh(core_axis_name="c", subcore_axis_name="s")

    @functools.partial(
        pl.kernel,
        mesh=mesh,
        compiler_params=_SC_PARAMS,
        out_type=jax.ShapeDtypeStruct((_N_PAD, d), jnp.float32),
        scratch_types=[
            pltpu.VMEM((_ROWS_W, d), jnp.float32),
            pltpu.VMEM((_BLK,), jnp.int32),
            pltpu.VMEM((_BLK,), jnp.int32),
            pltpu.VMEM((_BLK + _L,), jnp.int32),
            pltpu.VMEM((_BLK + _L,), jnp.int32),
            pltpu.VMEM((_BLK, d), jnp.float32),
            pltpu.VMEM((_BLK, d), jnp.float32),
            pltpu.VMEM((_L,), jnp.int32),
            pltpu.SemaphoreType.DMA,
            pltpu.SemaphoreType.DMA,
        ],
    )
    def fold_kernel(hp_hbm, lsrc_hbm, ldst_hbm, cnt_hbm, out_hbm,
                    acc, psrc_a, psrc_b, pdst_a, pdst_b, rows_a, rows_b,
                    crow, sem0, sem1):
        cid = lax.axis_index("c")
        sid = lax.axis_index("s")
        wid = cid * 16 + sid
        lo = wid * _ROWS_W

        fzero = jnp.zeros((_L,), jnp.float32)

        @pl.loop(0, _ROWS_W)
        def _(r):
            for c in range(d // _L):
                acc[r, pl.ds(c * _L, _L)] = fzero

        pltpu.sync_copy(cnt_hbm.at[wid], crow)
        total = crow[...][0]
        gmax = (total + _BLK - 1) // _BLK

        def issue(b, psrc, pdst, rows, sem):
            pltpu.sync_copy(lsrc_hbm.at[wid, pl.ds(b * _BLK, _BLK)], psrc)
            pltpu.sync_copy(ldst_hbm.at[wid, pl.ds(b * _BLK, _BLK)],
                            pdst.at[pl.ds(0, _BLK)])
            pltpu.async_copy(hp_hbm.at[psrc], rows, sem)

        def fold_block(b, psrc, pdst, rows, sem):
            pltpu.make_async_copy(hp_hbm.at[psrc], rows, sem).wait()
            nb = jnp.minimum(total - b * _BLK, _BLK)

            def fold(j, carry):
                ld = pdst[pl.ds(j, _L)][0]
                for c in range(d // _L):
                    sl = pl.ds(c * _L, _L)
                    acc[ld, sl] = jnp.maximum(acc[ld, sl], rows[j, sl])
                return carry

            lax.fori_loop(0, nb, fold, 0)

        @pl.when(gmax >= 1)
        def _():
            issue(0, psrc_a, pdst_a, rows_a, sem0)

        @pl.when(gmax >= 2)
        def _():
            issue(1, psrc_b, pdst_b, rows_b, sem1)

        @pl.loop(0, (gmax + 1) // 2)
        def _(t):
            b0 = 2 * t
            b1 = 2 * t + 1
            fold_block(b0, psrc_a, pdst_a, rows_a, sem0)

            @pl.when(b0 + 2 < gmax)
            def _():
                issue(b0 + 2, psrc_a, pdst_a, rows_a, sem0)

            @pl.when(b1 < gmax)
            def _():
                fold_block(b1, psrc_b, pdst_b, rows_b, sem1)

                @pl.when(b1 + 2 < gmax)
                def _():
                    issue(b1 + 2, psrc_b, pdst_b, rows_b, sem1)

        pltpu.sync_copy(acc, out_hbm.at[pl.ds(lo, _ROWS_W)])

    return fold_kernel(hp, lsrc, ldst, cnt)


def kernel(x, edge_index, Wp, bp, W1, W2):
    n, d = x.shape
    src = edge_index[0].astype(jnp.int32)
    dst = edge_index[1].astype(jnp.int32)
    e = src.shape[0]
    e_pad = ((e + _CHUNK - 1) // _CHUNK) * _CHUNK
    if e_pad != e:
        src = jnp.concatenate([src, jnp.zeros((e_pad - e,), jnp.int32)])
        dst = jnp.concatenate([dst, jnp.full((e_pad - e,), n, jnp.int32)])

    x_pad = jnp.zeros((_N_PAD, d), jnp.float32).at[:n].set(x)
    wpt = Wp.T
    bp8 = jnp.broadcast_to(bp.reshape(1, d), (8, d))
    w1t = W1.T
    w2t = W2.T
    wat1, wbt1 = w1t[:d], w1t[d:]
    wat2, wbt2 = w2t[:d], w2t[d:]

    lsrc, ldst, cnt = _edge_partition(src, dst)
    hp1 = _tc_pre(x_pad, wpt, bp8)
    ha1 = _fold(hp1, lsrc, ldst, cnt)
    h2, hp2 = _tc_post_hp(x_pad, ha1, wat1, wbt1, wpt, bp8)
    ha2 = _fold(hp2, lsrc, ldst, cnt)
    h3 = _tc_post(h2, ha2, wat2, wbt2)
    return h3[:n]


# speedup vs baseline: 1.9501x; 1.1213x over previous
import functools

import jax
import jax.numpy as jnp
from jax import lax
from jax.experimental import pallas as pl
from jax.experimental.pallas import tpu as pltpu
from jax.experimental.pallas import tpu_sc as plsc

_N_PAD = 10240
_NW = 32
_ROWS_W = _N_PAD // _NW
_CHUNK = 2000
_BLK = 64
_SUP = 4096
_BPS = _SUP // _BLK
_L = 16

_SC_PARAMS = pltpu.CompilerParams(needs_layout_passes=False)


def _dot(a, b):
    return jnp.dot(a, b, preferred_element_type=jnp.float32,
                   precision=lax.Precision.HIGHEST)


def _pre_body(h_ref, wpt_ref, bp_ref, o_ref):
    o_ref[...] = jax.nn.sigmoid(_dot(h_ref[...], wpt_ref[...]) + bp_ref[0:1, :])


def _tc_pre(h, wpt, bp8):
    d = h.shape[1]
    rb = 1024
    return pl.pallas_call(
        _pre_body,
        grid=(_N_PAD // rb,),
        in_specs=[
            pl.BlockSpec((rb, d), lambda i: (i, 0)),
            pl.BlockSpec((d, d), lambda i: (0, 0)),
            pl.BlockSpec((8, d), lambda i: (0, 0)),
        ],
        out_specs=pl.BlockSpec((rb, d), lambda i: (i, 0)),
        out_shape=jax.ShapeDtypeStruct((_N_PAD, d), jnp.float32),
    )(h, wpt, bp8)


def _post_update(h, ha, wat, wbt):
    s = jax.nn.sigmoid(_dot(h, wat) + _dot(ha, wbt))
    nrm = jnp.sqrt(jnp.sum(s * s, axis=1, keepdims=True))
    return s / jnp.maximum(nrm, 1e-12)


def _post_body(h_ref, ha_ref, wat_ref, wbt_ref, o_ref):
    o_ref[...] = _post_update(h_ref[...], ha_ref[...], wat_ref[...], wbt_ref[...])


def _post_hp_body(h_ref, ha_ref, wat_ref, wbt_ref, wpt_ref, bp_ref, o_ref,
                  ohp_ref):
    hn = _post_update(h_ref[...], ha_ref[...], wat_ref[...], wbt_ref[...])
    o_ref[...] = hn
    ohp_ref[...] = jax.nn.sigmoid(_dot(hn, wpt_ref[...]) + bp_ref[0:1, :])


def _tc_post(h, ha, wat, wbt):
    d = h.shape[1]
    rb = 1024
    return pl.pallas_call(
        _post_body,
        grid=(_N_PAD // rb,),
        in_specs=[
            pl.BlockSpec((rb, d), lambda i: (i, 0)),
            pl.BlockSpec((rb, d), lambda i: (i, 0)),
            pl.BlockSpec((d, d), lambda i: (0, 0)),
            pl.BlockSpec((d, d), lambda i: (0, 0)),
        ],
        out_specs=pl.BlockSpec((rb, d), lambda i: (i, 0)),
        out_shape=jax.ShapeDtypeStruct((_N_PAD, d), jnp.float32),
    )(h, ha, wat, wbt)


def _tc_post_hp(h, ha, wat, wbt, wpt, bp8):
    d = h.shape[1]
    rb = 1024
    return pl.pallas_call(
        _post_hp_body,
        grid=(_N_PAD // rb,),
        in_specs=[
            pl.BlockSpec((rb, d), lambda i: (i, 0)),
            pl.BlockSpec((rb, d), lambda i: (i, 0)),
            pl.BlockSpec((d, d), lambda i: (0, 0)),
            pl.BlockSpec((d, d), lambda i: (0, 0)),
            pl.BlockSpec((d, d), lambda i: (0, 0)),
            pl.BlockSpec((8, d), lambda i: (0, 0)),
        ],
        out_specs=[
            pl.BlockSpec((rb, d), lambda i: (i, 0)),
            pl.BlockSpec((rb, d), lambda i: (i, 0)),
        ],
        out_shape=[
            jax.ShapeDtypeStruct((_N_PAD, d), jnp.float32),
            jax.ShapeDtypeStruct((_N_PAD, d), jnp.float32),
        ],
    )(h, ha, wat, wbt, wpt, bp8)


def _edge_partition(src, dst):
    e_pad = src.shape[0]
    n_chunks = e_pad // _CHUNK
    n_pairs = (n_chunks + 1) // 2
    cap = ((e_pad + _SUP - 1) // _SUP) * _SUP
    mesh = plsc.VectorSubcoreMesh(core_axis_name="c", subcore_axis_name="s")

    @functools.partial(
        pl.kernel,
        mesh=mesh,
        compiler_params=_SC_PARAMS,
        out_type=[
            jax.ShapeDtypeStruct((_NW, cap), jnp.int32),
            jax.ShapeDtypeStruct((_NW, cap), jnp.int32),
            jax.ShapeDtypeStruct((_NW, _L), jnp.int32),
        ],
        scratch_types=[
            pltpu.VMEM((_CHUNK,), jnp.int32),
            pltpu.VMEM((_CHUNK,), jnp.int32),
            pltpu.VMEM((_CHUNK,), jnp.int32),
            pltpu.VMEM((_CHUNK,), jnp.int32),
            pltpu.VMEM((_BLK + _L,), jnp.int32),
            pltpu.VMEM((_BLK + _L,), jnp.int32),
            pltpu.SMEM((2,), jnp.int32),
            pltpu.VMEM((_L,), jnp.int32),
            pltpu.SemaphoreType.DMA,
            pltpu.SemaphoreType.DMA,
        ],
    )
    def part_kernel(src_hbm, dst_hbm, lsrc_hbm, ldst_hbm, cnt_hbm,
                    src_a, src_b, dst_a, dst_b, psrc, pdst, st, crow,
                    sem0, sem1):
        cid = lax.axis_index("c")
        sid = lax.axis_index("s")
        wid = cid * 16 + sid
        lo = wid * _ROWS_W
        hi = lo + _ROWS_W

        izero = jnp.zeros((_L,), jnp.int32)
        for c in range((_BLK + _L) // _L):
            psrc[pl.ds(c * _L, _L)] = izero
        st[0] = 0
        st[1] = 0

        def stage(cix, sbuf, dbuf, sem):
            base = cix * _CHUNK
            pltpu.async_copy(src_hbm.at[pl.ds(base, _CHUNK)], sbuf, sem)
            pltpu.async_copy(dst_hbm.at[pl.ds(base, _CHUNK)], dbuf, sem)

        def wait(cix, sbuf, dbuf, sem):
            pltpu.make_async_copy(src_hbm.at[pl.ds(cix * _CHUNK, _CHUNK)],
                                  sbuf, sem).wait()
            pltpu.make_async_copy(dst_hbm.at[pl.ds(cix * _CHUNK, _CHUNK)],
                                  dbuf, sem).wait()

        def emit_block():
            b = st[1]
            pltpu.sync_copy(psrc.at[pl.ds(0, _BLK)],
                            lsrc_hbm.at[wid, pl.ds(b * _BLK, _BLK)])
            pltpu.sync_copy(pdst.at[pl.ds(0, _BLK)],
                            ldst_hbm.at[wid, pl.ds(b * _BLK, _BLK)])
            st[1] = b + 1

        def scan(sbuf, dbuf):
            @pl.loop(0, _CHUNK, step=_L)
            def _(i):
                dvec = dbuf[pl.ds(i, _L)]
                svec = sbuf[pl.ds(i, _L)]
                m = (dvec >= lo) & (dvec < hi)
                k = st[0]
                plsc.store_compressed(psrc.at[pl.ds(k, _L)], svec, mask=m)
                plsc.store_compressed(pdst.at[pl.ds(k, _L)], dvec - lo, mask=m)
                st[0] = k + plsc.all_reduce_population_count(m)[0]

                @pl.when(st[0] >= _BLK)
                def _():
                    emit_block()
                    psrc[pl.ds(0, _L)] = psrc[pl.ds(_BLK, _L)]
                    pdst[pl.ds(0, _L)] = pdst[pl.ds(_BLK, _L)]
                    st[0] = st[0] - _BLK

        stage(0, src_a, dst_a, sem0)
        stage(1, src_b, dst_b, sem1)

        @pl.loop(0, n_pairs)
        def _(t):
            c0 = 2 * t
            c1 = 2 * t + 1
            wait(c0, src_a, dst_a, sem0)
            scan(src_a, dst_a)

            @pl.when(c0 + 2 < n_chunks)
            def _():
                stage(c0 + 2, src_a, dst_a, sem0)

            @pl.when(c1 < n_chunks)
            def _():
                wait(c1, src_b, dst_b, sem1)
                scan(src_b, dst_b)

                @pl.when(c1 + 2 < n_chunks)
                def _():
                    stage(c1 + 2, src_b, dst_b, sem1)

        total = st[1] * _BLK + st[0]

        @pl.when(st[0] > 0)
        def _():
            emit_block()

        crow[...] = jnp.full((_L,), total, jnp.int32)
        pltpu.sync_copy(crow, cnt_hbm.at[wid])

    return part_kernel(src, dst)


def _fold(hp, lsrc, ldst, cnt):
    d = hp.shape[1]
    mesh = plsc.VectorSubcoreMesh(core_axis_name="c", subcore_axis_name="s")

    @functools.partial(
        pl.kernel,
        mesh=mesh,
        compiler_params=_SC_PARAMS,
        out_type=jax.ShapeDtypeStruct((_N_PAD, d), jnp.float32),
        scratch_types=[
            pltpu.VMEM((_ROWS_W, d), jnp.float32),
            pltpu.VMEM((_SUP,), jnp.int32),
            pltpu.VMEM((_SUP + _L,), jnp.int32),
            pltpu.VMEM((_BLK, d), jnp.float32),
            pltpu.VMEM((_BLK, d), jnp.float32),
            pltpu.VMEM((_L,), jnp.int32),
            pltpu.SemaphoreType.DMA,
            pltpu.SemaphoreType.DMA,
        ],
    )
    def fold_kernel(hp_hbm, lsrc_hbm, ldst_hbm, cnt_hbm, out_hbm,
                    acc, sidx, didx, rows_a, rows_b, crow, sem0, sem1):
        cid = lax.axis_index("c")
        sid = lax.axis_index("s")
        wid = cid * 16 + sid
        lo = wid * _ROWS_W

        fzero = jnp.zeros((_L,), jnp.float32)

        @pl.loop(0, _ROWS_W)
        def _(r):
            for c in range(d // _L):
                acc[r, pl.ds(c * _L, _L)] = fzero

        pltpu.sync_copy(cnt_hbm.at[wid], crow)
        total = crow[...][0]
        gmax = (total + _BLK - 1) // _BLK
        n_sup = (total + _SUP - 1) // _SUP

        def idx_ref(b):
            return sidx.at[pl.ds(b * _BLK, _BLK)]

        def issue(b, rows, sem):
            pltpu.async_copy(hp_hbm.at[idx_ref(b)], rows, sem)

        def fold_block(s, b, rows, sem):
            pltpu.make_async_copy(hp_hbm.at[idx_ref(b)], rows, sem).wait()
            gb = s * _BPS + b
            nb = jnp.minimum(total - gb * _BLK, _BLK)
            base = b * _BLK

            def fold_row(j):
                ld = didx[pl.ds(base + j, _L)][0]
                for c in range(d // _L):
                    sl = pl.ds(c * _L, _L)
                    acc[ld, sl] = jnp.maximum(acc[ld, sl], rows[j, sl])

            @pl.when(nb == _BLK)
            def _():
                @pl.loop(0, _BLK, unroll=2)
                def _(j):
                    fold_row(j)

            @pl.when(nb < _BLK)
            def _():
                def fold(j, carry):
                    fold_row(j)
                    return carry

                lax.fori_loop(0, nb, fold, 0)

        @pl.loop(0, n_sup)
        def _(s):
            pltpu.sync_copy(lsrc_hbm.at[wid, pl.ds(s * _SUP, _SUP)], sidx)
            pltpu.sync_copy(ldst_hbm.at[wid, pl.ds(s * _SUP, _SUP)],
                            didx.at[pl.ds(0, _SUP)])
            nbh = jnp.minimum(gmax - s * _BPS, _BPS)

            @pl.when(nbh >= 1)
            def _():
                issue(0, rows_a, sem0)

            @pl.when(nbh >= 2)
            def _():
                issue(1, rows_b, sem1)

            @pl.loop(0, (nbh + 1) // 2)
            def _(t):
                b0 = 2 * t
                b1 = 2 * t + 1
                fold_block(s, b0, rows_a, sem0)

                @pl.when(b0 + 2 < nbh)
                def _():
                    issue(b0 + 2, rows_a, sem0)

                @pl.when(b1 < nbh)
                def _():
                    fold_block(s, b1, rows_b, sem1)

                    @pl.when(b1 + 2 < nbh)
                    def _():
                        issue(b1 + 2, rows_b, sem1)

        pltpu.sync_copy(acc, out_hbm.at[pl.ds(lo, _ROWS_W)])

    return fold_kernel(hp, lsrc, ldst, cnt)


def kernel(x, edge_index, Wp, bp, W1, W2):
    n, d = x.shape
    src = edge_index[0].astype(jnp.int32)
    dst = edge_index[1].astype(jnp.int32)
    e = src.shape[0]
    e_pad = ((e + _CHUNK - 1) // _CHUNK) * _CHUNK
    if e_pad != e:
        src = jnp.concatenate([src, jnp.zeros((e_pad - e,), jnp.int32)])
        dst = jnp.concatenate([dst, jnp.full((e_pad - e,), n, jnp.int32)])

    x_pad = jnp.zeros((_N_PAD, d), jnp.float32).at[:n].set(x)
    wpt = Wp.T
    bp8 = jnp.broadcast_to(bp.reshape(1, d), (8, d))
    w1t = W1.T
    w2t = W2.T
    wat1, wbt1 = w1t[:d], w1t[d:]
    wat2, wbt2 = w2t[:d], w2t[d:]

    lsrc, ldst, cnt = _edge_partition(src, dst)
    hp1 = _tc_pre(x_pad, wpt, bp8)
    ha1 = _fold(hp1, lsrc, ldst, cnt)
    h2, hp2 = _tc_post_hp(x_pad, ha1, wat1, wbt1, wpt, bp8)
    ha2 = _fold(hp2, lsrc, ldst, cnt)
    h3 = _tc_post(h2, ha2, wat2, wbt2)
    return h3[:n]


# speedup vs baseline: 2.9301x; 1.5025x over previous
import functools

import jax
import jax.numpy as jnp
from jax import lax
from jax.experimental import pallas as pl
from jax.experimental.pallas import tpu as pltpu
from jax.experimental.pallas import tpu_sc as plsc

_N_PAD = 10240
_NW = 32
_ROWS_W = _N_PAD // _NW
_CHUNK = 2000
_BLK = 128
_SUP = 4096
_BPS = _SUP // _BLK
_L = 16

_SC_PARAMS = pltpu.CompilerParams(needs_layout_passes=False)


def _dot(a, b):
    return jnp.dot(a, b, preferred_element_type=jnp.float32,
                   precision=lax.Precision.HIGHEST)


def _pre_body(h_ref, wpt_ref, bp_ref, o_ref):
    o_ref[...] = jax.nn.sigmoid(
        _dot(h_ref[...], wpt_ref[...]) + bp_ref[0:1, :]).astype(jnp.bfloat16)


def _tc_pre(h, wpt, bp8):
    d = h.shape[1]
    rb = 1024
    return pl.pallas_call(
        _pre_body,
        grid=(_N_PAD // rb,),
        in_specs=[
            pl.BlockSpec((rb, d), lambda i: (i, 0)),
            pl.BlockSpec((d, d), lambda i: (0, 0)),
            pl.BlockSpec((8, d), lambda i: (0, 0)),
        ],
        out_specs=pl.BlockSpec((rb, d), lambda i: (i, 0)),
        out_shape=jax.ShapeDtypeStruct((_N_PAD, d), jnp.bfloat16),
    )(h, wpt, bp8)


def _post_update(h, ha, wat, wbt):
    s = jax.nn.sigmoid(_dot(h, wat) + _dot(ha, wbt))
    nrm = jnp.sqrt(jnp.sum(s * s, axis=1, keepdims=True))
    return s / jnp.maximum(nrm, 1e-12)


def _post_body(h_ref, ha_ref, wat_ref, wbt_ref, o_ref):
    ha = ha_ref[...].astype(jnp.float32)
    o_ref[...] = _post_update(h_ref[...], ha, wat_ref[...], wbt_ref[...])


def _post_hp_body(h_ref, ha_ref, wat_ref, wbt_ref, wpt_ref, bp_ref, o_ref,
                  ohp_ref):
    ha = ha_ref[...].astype(jnp.float32)
    hn = _post_update(h_ref[...], ha, wat_ref[...], wbt_ref[...])
    o_ref[...] = hn
    ohp_ref[...] = jax.nn.sigmoid(
        _dot(hn, wpt_ref[...]) + bp_ref[0:1, :]).astype(jnp.bfloat16)


def _tc_post(h, ha, wat, wbt):
    d = h.shape[1]
    rb = 1024
    return pl.pallas_call(
        _post_body,
        grid=(_N_PAD // rb,),
        in_specs=[
            pl.BlockSpec((rb, d), lambda i: (i, 0)),
            pl.BlockSpec((rb, d), lambda i: (i, 0)),
            pl.BlockSpec((d, d), lambda i: (0, 0)),
            pl.BlockSpec((d, d), lambda i: (0, 0)),
        ],
        out_specs=pl.BlockSpec((rb, d), lambda i: (i, 0)),
        out_shape=jax.ShapeDtypeStruct((_N_PAD, d), jnp.float32),
    )(h, ha, wat, wbt)


def _tc_post_hp(h, ha, wat, wbt, wpt, bp8):
    d = h.shape[1]
    rb = 1024
    return pl.pallas_call(
        _post_hp_body,
        grid=(_N_PAD // rb,),
        in_specs=[
            pl.BlockSpec((rb, d), lambda i: (i, 0)),
            pl.BlockSpec((rb, d), lambda i: (i, 0)),
            pl.BlockSpec((d, d), lambda i: (0, 0)),
            pl.BlockSpec((d, d), lambda i: (0, 0)),
            pl.BlockSpec((d, d), lambda i: (0, 0)),
            pl.BlockSpec((8, d), lambda i: (0, 0)),
        ],
        out_specs=[
            pl.BlockSpec((rb, d), lambda i: (i, 0)),
            pl.BlockSpec((rb, d), lambda i: (i, 0)),
        ],
        out_shape=[
            jax.ShapeDtypeStruct((_N_PAD, d), jnp.float32),
            jax.ShapeDtypeStruct((_N_PAD, d), jnp.bfloat16),
        ],
    )(h, ha, wat, wbt, wpt, bp8)


def _edge_partition(src, dst):
    e_pad = src.shape[0]
    n_chunks = e_pad // _CHUNK
    n_pairs = (n_chunks + 1) // 2
    cap = ((e_pad + _SUP - 1) // _SUP) * _SUP
    mesh = plsc.VectorSubcoreMesh(core_axis_name="c", subcore_axis_name="s")

    @functools.partial(
        pl.kernel,
        mesh=mesh,
        compiler_params=_SC_PARAMS,
        out_type=[
            jax.ShapeDtypeStruct((_NW, cap), jnp.int32),
            jax.ShapeDtypeStruct((_NW, cap), jnp.int32),
            jax.ShapeDtypeStruct((_NW, _L), jnp.int32),
        ],
        scratch_types=[
            pltpu.VMEM((_CHUNK,), jnp.int32),
            pltpu.VMEM((_CHUNK,), jnp.int32),
            pltpu.VMEM((_CHUNK,), jnp.int32),
            pltpu.VMEM((_CHUNK,), jnp.int32),
            pltpu.VMEM((_BLK + _L,), jnp.int32),
            pltpu.VMEM((_BLK + _L,), jnp.int32),
            pltpu.SMEM((2,), jnp.int32),
            pltpu.VMEM((_L,), jnp.int32),
            pltpu.SemaphoreType.DMA,
            pltpu.SemaphoreType.DMA,
        ],
    )
    def part_kernel(src_hbm, dst_hbm, lsrc_hbm, ldst_hbm, cnt_hbm,
                    src_a, src_b, dst_a, dst_b, psrc, pdst, st, crow,
                    sem0, sem1):
        cid = lax.axis_index("c")
        sid = lax.axis_index("s")
        wid = cid * 16 + sid
        lo = wid * _ROWS_W
        hi = lo + _ROWS_W

        izero = jnp.zeros((_L,), jnp.int32)
        for c in range((_BLK + _L) // _L):
            psrc[pl.ds(c * _L, _L)] = izero
        st[0] = 0
        st[1] = 0

        def stage(cix, sbuf, dbuf, sem):
            base = cix * _CHUNK
            pltpu.async_copy(src_hbm.at[pl.ds(base, _CHUNK)], sbuf, sem)
            pltpu.async_copy(dst_hbm.at[pl.ds(base, _CHUNK)], dbuf, sem)

        def wait(cix, sbuf, dbuf, sem):
            pltpu.make_async_copy(src_hbm.at[pl.ds(cix * _CHUNK, _CHUNK)],
                                  sbuf, sem).wait()
            pltpu.make_async_copy(dst_hbm.at[pl.ds(cix * _CHUNK, _CHUNK)],
                                  dbuf, sem).wait()

        def emit_block():
            b = st[1]
            pltpu.sync_copy(psrc.at[pl.ds(0, _BLK)],
                            lsrc_hbm.at[wid, pl.ds(b * _BLK, _BLK)])
            pltpu.sync_copy(pdst.at[pl.ds(0, _BLK)],
                            ldst_hbm.at[wid, pl.ds(b * _BLK, _BLK)])
            st[1] = b + 1

        def scan(sbuf, dbuf):
            @pl.loop(0, _CHUNK, step=_L)
            def _(i):
                dvec = dbuf[pl.ds(i, _L)]
                svec = sbuf[pl.ds(i, _L)]
                m = (dvec >= lo) & (dvec < hi)
                k = st[0]
                plsc.store_compressed(psrc.at[pl.ds(k, _L)], svec, mask=m)
                plsc.store_compressed(pdst.at[pl.ds(k, _L)], dvec - lo, mask=m)
                st[0] = k + plsc.all_reduce_population_count(m)[0]

                @pl.when(st[0] >= _BLK)
                def _():
                    emit_block()
                    psrc[pl.ds(0, _L)] = psrc[pl.ds(_BLK, _L)]
                    pdst[pl.ds(0, _L)] = pdst[pl.ds(_BLK, _L)]
                    st[0] = st[0] - _BLK

        stage(0, src_a, dst_a, sem0)
        stage(1, src_b, dst_b, sem1)

        @pl.loop(0, n_pairs)
        def _(t):
            c0 = 2 * t
            c1 = 2 * t + 1
            wait(c0, src_a, dst_a, sem0)
            scan(src_a, dst_a)

            @pl.when(c0 + 2 < n_chunks)
            def _():
                stage(c0 + 2, src_a, dst_a, sem0)

            @pl.when(c1 < n_chunks)
            def _():
                wait(c1, src_b, dst_b, sem1)
                scan(src_b, dst_b)

                @pl.when(c1 + 2 < n_chunks)
                def _():
                    stage(c1 + 2, src_b, dst_b, sem1)

        total = st[1] * _BLK + st[0]

        @pl.when(st[0] > 0)
        def _():
            emit_block()

        crow[...] = jnp.full((_L,), total, jnp.int32)
        pltpu.sync_copy(crow, cnt_hbm.at[wid])

    return part_kernel(src, dst)


def _fold(hp, lsrc, ldst, cnt):
    d = 2 * hp.shape[1]
    mesh = plsc.VectorSubcoreMesh(core_axis_name="c", subcore_axis_name="s")

    @functools.partial(
        pl.kernel,
        mesh=mesh,
        compiler_params=_SC_PARAMS,
        out_type=jax.ShapeDtypeStruct((_N_PAD, d // 2), jnp.int32),
        scratch_types=[
            pltpu.VMEM((_ROWS_W, d // 2), jnp.int32),
            pltpu.VMEM((_SUP,), jnp.int32),
            pltpu.VMEM((_SUP + 2 * _L,), jnp.int32),
            pltpu.VMEM((_BLK, d // 2), jnp.int32),
            pltpu.VMEM((_BLK, d // 2), jnp.int32),
            pltpu.VMEM((_L,), jnp.int32),
            pltpu.SemaphoreType.DMA,
            pltpu.SemaphoreType.DMA,
        ],
    )
    def fold_kernel(hp_hbm, lsrc_hbm, ldst_hbm, cnt_hbm, out_hbm,
                    acc, sidx, didx, rows_a, rows_b, crow, sem0, sem1):
        cid = lax.axis_index("c")
        sid = lax.axis_index("s")
        wid = cid * 16 + sid
        lo = wid * _ROWS_W

        izero = jnp.zeros((_L,), jnp.int32)

        @pl.loop(0, _ROWS_W)
        def _(r):
            for c in range(d // (2 * _L)):
                acc[r, pl.ds(c * _L, _L)] = izero

        pltpu.sync_copy(cnt_hbm.at[wid], crow)
        total = crow[...][0]
        gmax = (total + _BLK - 1) // _BLK
        n_sup = (total + _SUP - 1) // _SUP

        def idx_ref(b):
            return sidx.at[pl.ds(b * _BLK, _BLK)]

        def issue(b, rows, sem):
            pltpu.async_copy(hp_hbm.at[idx_ref(b)], rows, sem)

        def fold_block(s, b, rows, sem):
            pltpu.make_async_copy(hp_hbm.at[idx_ref(b)], rows, sem).wait()
            gb = s * _BPS + b
            nb = jnp.minimum(total - gb * _BLK, _BLK)
            base = b * _BLK
            w = 2 * _L

            def fold_row(j, ld):
                ld_next = didx[pl.ds(base + j + 1, _L)][0]
                for c in range(0, d // w, 2):
                    sl0 = pl.ds(c * _L, _L)
                    sl1 = pl.ds((c + 1) * _L, _L)
                    a0 = plsc.bitcast(acc[ld, sl0], jnp.bfloat16)
                    r0 = plsc.bitcast(rows[j, sl0], jnp.bfloat16)
                    a1 = plsc.bitcast(acc[ld, sl1], jnp.bfloat16)
                    r1 = plsc.bitcast(rows[j, sl1], jnp.bfloat16)
                    acc[ld, sl0] = plsc.bitcast(jnp.maximum(a0, r0),
                                                jnp.int32)
                    acc[ld, sl1] = plsc.bitcast(jnp.maximum(a1, r1),
                                                jnp.int32)
                return ld_next

            ld0 = didx[pl.ds(base, _L)][0]

            @pl.when(nb == _BLK)
            def _():
                lax.fori_loop(0, _BLK, fold_row, ld0, unroll=2)

            @pl.when(nb < _BLK)
            def _():
                lax.fori_loop(0, nb, fold_row, ld0)

        @pl.loop(0, n_sup)
        def _(s):
            pltpu.sync_copy(lsrc_hbm.at[wid, pl.ds(s * _SUP, _SUP)], sidx)
            pltpu.sync_copy(ldst_hbm.at[wid, pl.ds(s * _SUP, _SUP)],
                            didx.at[pl.ds(0, _SUP)])
            nbh = jnp.minimum(gmax - s * _BPS, _BPS)

            @pl.when(nbh >= 1)
            def _():
                issue(0, rows_a, sem0)

            @pl.when(nbh >= 2)
            def _():
                issue(1, rows_b, sem1)

            @pl.loop(0, (nbh + 1) // 2)
            def _(t):
                b0 = 2 * t
                b1 = 2 * t + 1
                fold_block(s, b0, rows_a, sem0)

                @pl.when(b0 + 2 < nbh)
                def _():
                    issue(b0 + 2, rows_a, sem0)

                @pl.when(b1 < nbh)
                def _():
                    fold_block(s, b1, rows_b, sem1)

                    @pl.when(b1 + 2 < nbh)
                    def _():
                        issue(b1 + 2, rows_b, sem1)

        pltpu.sync_copy(acc, out_hbm.at[pl.ds(lo, _ROWS_W)])

    return fold_kernel(hp, lsrc, ldst, cnt)


def kernel(x, edge_index, Wp, bp, W1, W2):
    n, d = x.shape
    src = edge_index[0].astype(jnp.int32)
    dst = edge_index[1].astype(jnp.int32)
    e = src.shape[0]
    e_pad = ((e + _CHUNK - 1) // _CHUNK) * _CHUNK
    if e_pad != e:
        src = jnp.concatenate([src, jnp.zeros((e_pad - e,), jnp.int32)])
        dst = jnp.concatenate([dst, jnp.full((e_pad - e,), n, jnp.int32)])

    x_pad = jnp.zeros((_N_PAD, d), jnp.float32).at[:n].set(x)
    wpt = Wp.T
    bp8 = jnp.broadcast_to(bp.reshape(1, d), (8, d))
    w1t = W1.T
    w2t = W2.T
    wat1, wbt1 = w1t[:d], w1t[d:]
    wat2, wbt2 = w2t[:d], w2t[d:]

    def as_i32(hp):
        return lax.bitcast_convert_type(
            hp.reshape(_N_PAD, d // 2, 2), jnp.int32)

    def as_bf16(ha_i32):
        return lax.bitcast_convert_type(
            ha_i32, jnp.bfloat16).reshape(_N_PAD, d)

    lsrc, ldst, cnt = _edge_partition(src, dst)
    hp1 = _tc_pre(x_pad, wpt, bp8)
    ha1 = _fold(as_i32(hp1), lsrc, ldst, cnt)
    h2, hp2 = _tc_post_hp(x_pad, as_bf16(ha1), wat1, wbt1, wpt, bp8)
    ha2 = _fold(as_i32(hp2), lsrc, ldst, cnt)
    h3 = _tc_post(h2, as_bf16(ha2), wat2, wbt2)
    return h3[:n]


# speedup vs baseline: 3.0540x; 1.0423x over previous
import functools

import jax
import jax.numpy as jnp
from jax import lax
from jax.experimental import pallas as pl
from jax.experimental.pallas import tpu as pltpu
from jax.experimental.pallas import tpu_sc as plsc

_N_PAD = 10240
_NW = 32
_ROWS_W = _N_PAD // _NW
_CHUNK = 2000
_BLK = 128
_SUP = 4096
_BPS = _SUP // _BLK
_L = 16
_PCAP = _CHUNK + 2 * _BLK

_SC_PARAMS = pltpu.CompilerParams(needs_layout_passes=False)


def _dot(a, b):
    return jnp.dot(a, b, preferred_element_type=jnp.float32,
                   precision=lax.Precision.HIGHEST)


def _pre_body(h_ref, wpt_ref, bp_ref, o_ref):
    o_ref[...] = jax.nn.sigmoid(
        _dot(h_ref[...], wpt_ref[...]) + bp_ref[0:1, :]).astype(jnp.bfloat16)


def _tc_pre(h, wpt, bp8):
    d = h.shape[1]
    rb = 1024
    return pl.pallas_call(
        _pre_body,
        grid=(_N_PAD // rb,),
        in_specs=[
            pl.BlockSpec((rb, d), lambda i: (i, 0)),
            pl.BlockSpec((d, d), lambda i: (0, 0)),
            pl.BlockSpec((8, d), lambda i: (0, 0)),
        ],
        out_specs=pl.BlockSpec((rb, d), lambda i: (i, 0)),
        out_shape=jax.ShapeDtypeStruct((_N_PAD, d), jnp.bfloat16),
    )(h, wpt, bp8)


def _post_update(h, ha, wat, wbt):
    s = jax.nn.sigmoid(_dot(h, wat) + _dot(ha, wbt))
    nrm = jnp.sqrt(jnp.sum(s * s, axis=1, keepdims=True))
    return s / jnp.maximum(nrm, 1e-12)


def _post_body(h_ref, ha_ref, wat_ref, wbt_ref, o_ref):
    ha = ha_ref[...].astype(jnp.float32)
    o_ref[...] = _post_update(h_ref[...], ha, wat_ref[...], wbt_ref[...])


def _post_hp_body(h_ref, ha_ref, wat_ref, wbt_ref, wpt_ref, bp_ref, o_ref,
                  ohp_ref):
    ha = ha_ref[...].astype(jnp.float32)
    hn = _post_update(h_ref[...], ha, wat_ref[...], wbt_ref[...])
    o_ref[...] = hn
    ohp_ref[...] = jax.nn.sigmoid(
        _dot(hn, wpt_ref[...]) + bp_ref[0:1, :]).astype(jnp.bfloat16)


def _tc_post(h, ha, wat, wbt):
    d = h.shape[1]
    rb = 1024
    return pl.pallas_call(
        _post_body,
        grid=(_N_PAD // rb,),
        in_specs=[
            pl.BlockSpec((rb, d), lambda i: (i, 0)),
            pl.BlockSpec((rb, d), lambda i: (i, 0)),
            pl.BlockSpec((d, d), lambda i: (0, 0)),
            pl.BlockSpec((d, d), lambda i: (0, 0)),
        ],
        out_specs=pl.BlockSpec((rb, d), lambda i: (i, 0)),
        out_shape=jax.ShapeDtypeStruct((_N_PAD, d), jnp.float32),
    )(h, ha, wat, wbt)


def _tc_post_hp(h, ha, wat, wbt, wpt, bp8):
    d = h.shape[1]
    rb = 1024
    return pl.pallas_call(
        _post_hp_body,
        grid=(_N_PAD // rb,),
        in_specs=[
            pl.BlockSpec((rb, d), lambda i: (i, 0)),
            pl.BlockSpec((rb, d), lambda i: (i, 0)),
            pl.BlockSpec((d, d), lambda i: (0, 0)),
            pl.BlockSpec((d, d), lambda i: (0, 0)),
            pl.BlockSpec((d, d), lambda i: (0, 0)),
            pl.BlockSpec((8, d), lambda i: (0, 0)),
        ],
        out_specs=[
            pl.BlockSpec((rb, d), lambda i: (i, 0)),
            pl.BlockSpec((rb, d), lambda i: (i, 0)),
        ],
        out_shape=[
            jax.ShapeDtypeStruct((_N_PAD, d), jnp.float32),
            jax.ShapeDtypeStruct((_N_PAD, d), jnp.bfloat16),
        ],
    )(h, ha, wat, wbt, wpt, bp8)


def _edge_partition(src, dst):
    e_pad = src.shape[0]
    n_chunks = e_pad // _CHUNK
    n_pairs = (n_chunks + 1) // 2
    cap = ((e_pad + _SUP - 1) // _SUP) * _SUP
    mesh = plsc.VectorSubcoreMesh(core_axis_name="c", subcore_axis_name="s")

    @functools.partial(
        pl.kernel,
        mesh=mesh,
        compiler_params=_SC_PARAMS,
        out_type=[
            jax.ShapeDtypeStruct((_NW, cap), jnp.int32),
            jax.ShapeDtypeStruct((_NW, cap), jnp.int32),
            jax.ShapeDtypeStruct((_NW, _L), jnp.int32),
        ],
        scratch_types=[
            pltpu.VMEM((_CHUNK,), jnp.int32),
            pltpu.VMEM((_CHUNK,), jnp.int32),
            pltpu.VMEM((_CHUNK,), jnp.int32),
            pltpu.VMEM((_CHUNK,), jnp.int32),
            pltpu.VMEM((_PCAP,), jnp.int32),
            pltpu.VMEM((_PCAP,), jnp.int32),
            pltpu.SMEM((2,), jnp.int32),
            pltpu.VMEM((_L,), jnp.int32),
            pltpu.SemaphoreType.DMA,
            pltpu.SemaphoreType.DMA,
        ],
    )
    def part_kernel(src_hbm, dst_hbm, lsrc_hbm, ldst_hbm, cnt_hbm,
                    src_a, src_b, dst_a, dst_b, psrc, pdst, st, crow,
                    sem0, sem1):
        cid = lax.axis_index("c")
        sid = lax.axis_index("s")
        wid = cid * 16 + sid
        lo = wid * _ROWS_W
        hi = lo + _ROWS_W

        izero = jnp.zeros((_L,), jnp.int32)

        @pl.loop(0, _PCAP, step=_L)
        def _(i):
            psrc[pl.ds(i, _L)] = izero

        st[0] = 0
        st[1] = 0

        def stage(cix, sbuf, dbuf, sem):
            base = cix * _CHUNK
            pltpu.async_copy(src_hbm.at[pl.ds(base, _CHUNK)], sbuf, sem)
            pltpu.async_copy(dst_hbm.at[pl.ds(base, _CHUNK)], dbuf, sem)

        def wait(cix, sbuf, dbuf, sem):
            pltpu.make_async_copy(src_hbm.at[pl.ds(cix * _CHUNK, _CHUNK)],
                                  sbuf, sem).wait()
            pltpu.make_async_copy(dst_hbm.at[pl.ds(cix * _CHUNK, _CHUNK)],
                                  dbuf, sem).wait()

        def scan(sbuf, dbuf):
            kv = jnp.full((_L,), st[0] - 1, jnp.int32)

            def group(g, kv):
                i = g * _L
                dvec = dbuf[pl.ds(i, _L)]
                svec = sbuf[pl.ds(i, _L)]
                m = (dvec >= lo) & (dvec < hi)
                pos = kv + jnp.cumsum(m.astype(jnp.int32))
                plsc.store_scatter(psrc, [pos], svec, mask=m)
                plsc.store_scatter(pdst, [pos], dvec - lo, mask=m)
                return kv + plsc.all_reduce_population_count(m)

            kv = lax.fori_loop(0, _CHUNK // _L, group, kv, unroll=4)
            k = kv[0] + 1
            nb = k // _BLK

            def emit(b, carry):
                bo = st[1] + b
                pltpu.sync_copy(
                    psrc.at[pl.ds(b * _BLK, _BLK)],
                    lsrc_hbm.at[wid, pl.ds(bo * _BLK, _BLK)])
                pltpu.sync_copy(
                    pdst.at[pl.ds(b * _BLK, _BLK)],
                    ldst_hbm.at[wid, pl.ds(bo * _BLK, _BLK)])
                return carry

            lax.fori_loop(0, nb, emit, 0)
            st[1] = st[1] + nb
            base = nb * _BLK

            @pl.when(nb > 0)
            def _():
                for c in range(_BLK // _L):
                    off = c * _L
                    psrc[pl.ds(off, _L)] = psrc[pl.ds(base + off, _L)]
                    pdst[pl.ds(off, _L)] = pdst[pl.ds(base + off, _L)]

            st[0] = k - base

        stage(0, src_a, dst_a, sem0)
        stage(1, src_b, dst_b, sem1)

        @pl.loop(0, n_pairs)
        def _(t):
            c0 = 2 * t
            c1 = 2 * t + 1
            wait(c0, src_a, dst_a, sem0)
            scan(src_a, dst_a)

            @pl.when(c0 + 2 < n_chunks)
            def _():
                stage(c0 + 2, src_a, dst_a, sem0)

            @pl.when(c1 < n_chunks)
            def _():
                wait(c1, src_b, dst_b, sem1)
                scan(src_b, dst_b)

                @pl.when(c1 + 2 < n_chunks)
                def _():
                    stage(c1 + 2, src_b, dst_b, sem1)

        total = st[1] * _BLK + st[0]

        @pl.when(st[0] > 0)
        def _():
            b = st[1]
            pltpu.sync_copy(psrc.at[pl.ds(0, _BLK)],
                            lsrc_hbm.at[wid, pl.ds(b * _BLK, _BLK)])
            pltpu.sync_copy(pdst.at[pl.ds(0, _BLK)],
                            ldst_hbm.at[wid, pl.ds(b * _BLK, _BLK)])

        crow[...] = jnp.full((_L,), total, jnp.int32)
        pltpu.sync_copy(crow, cnt_hbm.at[wid])

    return part_kernel(src, dst)


def _fold(hp, lsrc, ldst, cnt):
    d = 2 * hp.shape[1]
    mesh = plsc.VectorSubcoreMesh(core_axis_name="c", subcore_axis_name="s")

    @functools.partial(
        pl.kernel,
        mesh=mesh,
        compiler_params=_SC_PARAMS,
        out_type=jax.ShapeDtypeStruct((_N_PAD, d // 2), jnp.int32),
        scratch_types=[
            pltpu.VMEM((_ROWS_W, d // 2), jnp.int32),
            pltpu.VMEM((_SUP,), jnp.int32),
            pltpu.VMEM((_SUP + 2 * _L,), jnp.int32),
            pltpu.VMEM((_BLK, d // 2), jnp.int32),
            pltpu.VMEM((_BLK, d // 2), jnp.int32),
            pltpu.VMEM((_L,), jnp.int32),
            pltpu.SemaphoreType.DMA,
            pltpu.SemaphoreType.DMA,
        ],
    )
    def fold_kernel(hp_hbm, lsrc_hbm, ldst_hbm, cnt_hbm, out_hbm,
                    acc, sidx, didx, rows_a, rows_b, crow, sem0, sem1):
        cid = lax.axis_index("c")
        sid = lax.axis_index("s")
        wid = cid * 16 + sid
        lo = wid * _ROWS_W

        izero = jnp.zeros((_L,), jnp.int32)

        @pl.loop(0, _ROWS_W)
        def _(r):
            for c in range(d // (2 * _L)):
                acc[r, pl.ds(c * _L, _L)] = izero

        pltpu.sync_copy(cnt_hbm.at[wid], crow)
        total = crow[...][0]
        gmax = (total + _BLK - 1) // _BLK
        n_sup = (total + _SUP - 1) // _SUP

        def idx_ref(b):
            return sidx.at[pl.ds(b * _BLK, _BLK)]

        def issue(b, rows, sem):
            pltpu.async_copy(hp_hbm.at[idx_ref(b)], rows, sem)

        def fold_block(s, b, rows, sem):
            pltpu.make_async_copy(hp_hbm.at[idx_ref(b)], rows, sem).wait()
            gb = s * _BPS + b
            nb = jnp.minimum(total - gb * _BLK, _BLK)
            base = b * _BLK
            w = 2 * _L

            def fold_row(j, ld):
                ld_next = didx[pl.ds(base + j + 1, _L)][0]
                for c in range(0, d // w, 2):
                    sl0 = pl.ds(c * _L, _L)
                    sl1 = pl.ds((c + 1) * _L, _L)
                    a0 = plsc.bitcast(acc[ld, sl0], jnp.bfloat16)
                    r0 = plsc.bitcast(rows[j, sl0], jnp.bfloat16)
                    a1 = plsc.bitcast(acc[ld, sl1], jnp.bfloat16)
                    r1 = plsc.bitcast(rows[j, sl1], jnp.bfloat16)
                    acc[ld, sl0] = plsc.bitcast(jnp.maximum(a0, r0),
                                                jnp.int32)
                    acc[ld, sl1] = plsc.bitcast(jnp.maximum(a1, r1),
                                                jnp.int32)
                return ld_next

            ld0 = didx[pl.ds(base, _L)][0]

            @pl.when(nb == _BLK)
            def _():
                lax.fori_loop(0, _BLK, fold_row, ld0, unroll=2)

            @pl.when(nb < _BLK)
            def _():
                lax.fori_loop(0, nb, fold_row, ld0)

        @pl.loop(0, n_sup)
        def _(s):
            pltpu.sync_copy(lsrc_hbm.at[wid, pl.ds(s * _SUP, _SUP)], sidx)
            pltpu.sync_copy(ldst_hbm.at[wid, pl.ds(s * _SUP, _SUP)],
                            didx.at[pl.ds(0, _SUP)])
            nbh = jnp.minimum(gmax - s * _BPS, _BPS)

            @pl.when(nbh >= 1)
            def _():
                issue(0, rows_a, sem0)

            @pl.when(nbh >= 2)
            def _():
                issue(1, rows_b, sem1)

            @pl.loop(0, (nbh + 1) // 2)
            def _(t):
                b0 = 2 * t
                b1 = 2 * t + 1
                fold_block(s, b0, rows_a, sem0)

                @pl.when(b0 + 2 < nbh)
                def _():
                    issue(b0 + 2, rows_a, sem0)

                @pl.when(b1 < nbh)
                def _():
                    fold_block(s, b1, rows_b, sem1)

                    @pl.when(b1 + 2 < nbh)
                    def _():
                        issue(b1 + 2, rows_b, sem1)

        pltpu.sync_copy(acc, out_hbm.at[pl.ds(lo, _ROWS_W)])

    return fold_kernel(hp, lsrc, ldst, cnt)


def kernel(x, edge_index, Wp, bp, W1, W2):
    n, d = x.shape
    src = edge_index[0].astype(jnp.int32)
    dst = edge_index[1].astype(jnp.int32)
    e = src.shape[0]
    e_pad = ((e + _CHUNK - 1) // _CHUNK) * _CHUNK
    if e_pad != e:
        src = jnp.concatenate([src, jnp.zeros((e_pad - e,), jnp.int32)])
        dst = jnp.concatenate([dst, jnp.full((e_pad - e,), n, jnp.int32)])

    x_pad = jnp.zeros((_N_PAD, d), jnp.float32).at[:n].set(x)
    wpt = Wp.T
    bp8 = jnp.broadcast_to(bp.reshape(1, d), (8, d))
    w1t = W1.T
    w2t = W2.T
    wat1, wbt1 = w1t[:d], w1t[d:]
    wat2, wbt2 = w2t[:d], w2t[d:]

    def as_i32(hp):
        return lax.bitcast_convert_type(
            hp.reshape(_N_PAD, d // 2, 2), jnp.int32)

    def as_bf16(ha_i32):
        return lax.bitcast_convert_type(
            ha_i32, jnp.bfloat16).reshape(_N_PAD, d)

    lsrc, ldst, cnt = _edge_partition(src, dst)
    hp1 = _tc_pre(x_pad, wpt, bp8)
    ha1 = _fold(as_i32(hp1), lsrc, ldst, cnt)
    h2, hp2 = _tc_post_hp(x_pad, as_bf16(ha1), wat1, wbt1, wpt, bp8)
    ha2 = _fold(as_i32(hp2), lsrc, ldst, cnt)
    h3 = _tc_post(h2, as_bf16(ha2), wat2, wbt2)
    return h3[:n]


# speedup vs baseline: 4.1459x; 1.3575x over previous
import functools

import jax
import jax.numpy as jnp
from jax import lax
from jax.experimental import pallas as pl
from jax.experimental.pallas import tpu as pltpu
from jax.experimental.pallas import tpu_sc as plsc

_N_PAD = 10240
_NW = 32
_ROWS_W = _N_PAD // _NW
_CHUNK = 2000
_BLK = 128
_SUP = 4096
_BPS = _SUP // _BLK
_L = 16
_PCAP = _CHUNK + 2 * _BLK

_SC_PARAMS = pltpu.CompilerParams(needs_layout_passes=False)


def _dot(a, b):
    return jnp.dot(a, b, preferred_element_type=jnp.float32,
                   precision=lax.Precision.HIGHEST)


def _pack_bf16(s):
    k = s.shape[1] // 2
    def bits(x):
        b = lax.bitcast_convert_type(x, jnp.int32)
        rnd = (lax.shift_right_logical(b, 16) & 1) + 0x7FFF
        return lax.shift_right_logical(b + rnd, 16)
    lo = bits(s[:, :k])
    hi = bits(s[:, k:])
    return lo | lax.shift_left(hi, 16)


def _unpack_bf16(w):
    lo = lax.bitcast_convert_type(lax.shift_left(w, 16), jnp.float32)
    hi = lax.bitcast_convert_type(w & jnp.int32(-65536), jnp.float32)
    return lo, hi


def _pre_body(h_ref, wpt_ref, bp_ref, o_ref):
    o_ref[...] = _pack_bf16(
        jax.nn.sigmoid(_dot(h_ref[...], wpt_ref[...]) + bp_ref[0:1, :]))


def _tc_pre(h, wpt, bp8):
    d = h.shape[1]
    rb = 1024
    return pl.pallas_call(
        _pre_body,
        grid=(_N_PAD // rb,),
        in_specs=[
            pl.BlockSpec((rb, d), lambda i: (i, 0)),
            pl.BlockSpec((d, d), lambda i: (0, 0)),
            pl.BlockSpec((8, d), lambda i: (0, 0)),
        ],
        out_specs=pl.BlockSpec((rb, d // 2), lambda i: (i, 0)),
        out_shape=jax.ShapeDtypeStruct((_N_PAD, d // 2), jnp.int32),
    )(h, wpt, bp8)


def _post_update(h, ha_words, wat, wbt_lo, wbt_hi):
    ha_lo, ha_hi = _unpack_bf16(ha_words)
    s = jax.nn.sigmoid(_dot(h, wat) + _dot(ha_lo, wbt_lo)
                       + _dot(ha_hi, wbt_hi))
    nrm = jnp.sqrt(jnp.sum(s * s, axis=1, keepdims=True))
    return s / jnp.maximum(nrm, 1e-12)


def _post_body(h_ref, ha_ref, wat_ref, wbl_ref, wbh_ref, o_ref):
    o_ref[...] = _post_update(h_ref[...], ha_ref[...], wat_ref[...],
                              wbl_ref[...], wbh_ref[...])


def _post_hp_body(h_ref, ha_ref, wat_ref, wbl_ref, wbh_ref, wpt_ref, bp_ref,
                  o_ref, ohp_ref):
    hn = _post_update(h_ref[...], ha_ref[...], wat_ref[...], wbl_ref[...],
                      wbh_ref[...])
    o_ref[...] = hn
    ohp_ref[...] = _pack_bf16(
        jax.nn.sigmoid(_dot(hn, wpt_ref[...]) + bp_ref[0:1, :]))


def _tc_post(h, ha, wat, wbt_lo, wbt_hi):
    d = h.shape[1]
    rb = 1024
    return pl.pallas_call(
        _post_body,
        grid=(_N_PAD // rb,),
        in_specs=[
            pl.BlockSpec((rb, d), lambda i: (i, 0)),
            pl.BlockSpec((rb, d // 2), lambda i: (i, 0)),
            pl.BlockSpec((d, d), lambda i: (0, 0)),
            pl.BlockSpec((d // 2, d), lambda i: (0, 0)),
            pl.BlockSpec((d // 2, d), lambda i: (0, 0)),
        ],
        out_specs=pl.BlockSpec((rb, d), lambda i: (i, 0)),
        out_shape=jax.ShapeDtypeStruct((_N_PAD, d), jnp.float32),
    )(h, ha, wat, wbt_lo, wbt_hi)


def _tc_post_hp(h, ha, wat, wbt_lo, wbt_hi, wpt, bp8):
    d = h.shape[1]
    rb = 1024
    return pl.pallas_call(
        _post_hp_body,
        grid=(_N_PAD // rb,),
        in_specs=[
            pl.BlockSpec((rb, d), lambda i: (i, 0)),
            pl.BlockSpec((rb, d // 2), lambda i: (i, 0)),
            pl.BlockSpec((d, d), lambda i: (0, 0)),
            pl.BlockSpec((d // 2, d), lambda i: (0, 0)),
            pl.BlockSpec((d // 2, d), lambda i: (0, 0)),
            pl.BlockSpec((d, d), lambda i: (0, 0)),
            pl.BlockSpec((8, d), lambda i: (0, 0)),
        ],
        out_specs=[
            pl.BlockSpec((rb, d), lambda i: (i, 0)),
            pl.BlockSpec((rb, d // 2), lambda i: (i, 0)),
        ],
        out_shape=[
            jax.ShapeDtypeStruct((_N_PAD, d), jnp.float32),
            jax.ShapeDtypeStruct((_N_PAD, d // 2), jnp.int32),
        ],
    )(h, ha, wat, wbt_lo, wbt_hi, wpt, bp8)


def _edge_partition(src, dst):
    e_pad = src.shape[0]
    n_chunks = e_pad // _CHUNK
    n_pairs = (n_chunks + 1) // 2
    cap = ((e_pad + _SUP - 1) // _SUP) * _SUP
    mesh = plsc.VectorSubcoreMesh(core_axis_name="c", subcore_axis_name="s")

    @functools.partial(
        pl.kernel,
        mesh=mesh,
        compiler_params=_SC_PARAMS,
        out_type=[
            jax.ShapeDtypeStruct((_NW, cap), jnp.int32),
            jax.ShapeDtypeStruct((_NW, cap), jnp.int32),
            jax.ShapeDtypeStruct((_NW, _L), jnp.int32),
        ],
        scratch_types=[
            pltpu.VMEM((_CHUNK,), jnp.int32),
            pltpu.VMEM((_CHUNK,), jnp.int32),
            pltpu.VMEM((_CHUNK,), jnp.int32),
            pltpu.VMEM((_CHUNK,), jnp.int32),
            pltpu.VMEM((_PCAP,), jnp.int32),
            pltpu.VMEM((_PCAP,), jnp.int32),
            pltpu.SMEM((2,), jnp.int32),
            pltpu.VMEM((_L,), jnp.int32),
            pltpu.SemaphoreType.DMA,
            pltpu.SemaphoreType.DMA,
        ],
    )
    def part_kernel(src_hbm, dst_hbm, lsrc_hbm, ldst_hbm, cnt_hbm,
                    src_a, src_b, dst_a, dst_b, psrc, pdst, st, crow,
                    sem0, sem1):
        cid = lax.axis_index("c")
        sid = lax.axis_index("s")
        wid = cid * 16 + sid
        lo = wid * _ROWS_W
        hi = lo + _ROWS_W

        izero = jnp.zeros((_L,), jnp.int32)

        @pl.loop(0, _PCAP, step=_L)
        def _(i):
            psrc[pl.ds(i, _L)] = izero

        st[0] = 0
        st[1] = 0

        def stage(cix, sbuf, dbuf, sem):
            base = cix * _CHUNK
            pltpu.async_copy(src_hbm.at[pl.ds(base, _CHUNK)], sbuf, sem)
            pltpu.async_copy(dst_hbm.at[pl.ds(base, _CHUNK)], dbuf, sem)

        def wait(cix, sbuf, dbuf, sem):
            pltpu.make_async_copy(src_hbm.at[pl.ds(cix * _CHUNK, _CHUNK)],
                                  sbuf, sem).wait()
            pltpu.make_async_copy(dst_hbm.at[pl.ds(cix * _CHUNK, _CHUNK)],
                                  dbuf, sem).wait()

        def scan(sbuf, dbuf):
            kv = jnp.full((_L,), st[0] - 1, jnp.int32)

            def group(g, kv):
                i = g * _L
                dvec = dbuf[pl.ds(i, _L)]
                svec = sbuf[pl.ds(i, _L)]
                m = (dvec >= lo) & (dvec < hi)
                pos = kv + jnp.cumsum(m.astype(jnp.int32))
                plsc.store_scatter(psrc, [pos], svec, mask=m)
                plsc.store_scatter(pdst, [pos], dvec - lo, mask=m)
                return kv + plsc.all_reduce_population_count(m)

            kv = lax.fori_loop(0, _CHUNK // _L, group, kv, unroll=4)
            k = kv[0] + 1
            nb = k // _BLK

            def emit(b, carry):
                bo = st[1] + b
                pltpu.sync_copy(
                    psrc.at[pl.ds(b * _BLK, _BLK)],
                    lsrc_hbm.at[wid, pl.ds(bo * _BLK, _BLK)])
                pltpu.sync_copy(
                    pdst.at[pl.ds(b * _BLK, _BLK)],
                    ldst_hbm.at[wid, pl.ds(bo * _BLK, _BLK)])
                return carry

            lax.fori_loop(0, nb, emit, 0)
            st[1] = st[1] + nb
            base = nb * _BLK

            @pl.when(nb > 0)
            def _():
                for c in range(_BLK // _L):
                    off = c * _L
                    psrc[pl.ds(off, _L)] = psrc[pl.ds(base + off, _L)]
                    pdst[pl.ds(off, _L)] = pdst[pl.ds(base + off, _L)]

            st[0] = k - base

        stage(0, src_a, dst_a, sem0)
        stage(1, src_b, dst_b, sem1)

        @pl.loop(0, n_pairs)
        def _(t):
            c0 = 2 * t
            c1 = 2 * t + 1
            wait(c0, src_a, dst_a, sem0)
            scan(src_a, dst_a)

            @pl.when(c0 + 2 < n_chunks)
            def _():
                stage(c0 + 2, src_a, dst_a, sem0)

            @pl.when(c1 < n_chunks)
            def _():
                wait(c1, src_b, dst_b, sem1)
                scan(src_b, dst_b)

                @pl.when(c1 + 2 < n_chunks)
                def _():
                    stage(c1 + 2, src_b, dst_b, sem1)

        total = st[1] * _BLK + st[0]

        @pl.when(st[0] > 0)
        def _():
            b = st[1]
            pltpu.sync_copy(psrc.at[pl.ds(0, _BLK)],
                            lsrc_hbm.at[wid, pl.ds(b * _BLK, _BLK)])
            pltpu.sync_copy(pdst.at[pl.ds(0, _BLK)],
                            ldst_hbm.at[wid, pl.ds(b * _BLK, _BLK)])

        crow[...] = jnp.full((_L,), total, jnp.int32)
        pltpu.sync_copy(crow, cnt_hbm.at[wid])

    return part_kernel(src, dst)


def _fold(hp, lsrc, ldst, cnt):
    d = 2 * hp.shape[1]
    mesh = plsc.VectorSubcoreMesh(core_axis_name="c", subcore_axis_name="s")

    @functools.partial(
        pl.kernel,
        mesh=mesh,
        compiler_params=_SC_PARAMS,
        out_type=jax.ShapeDtypeStruct((_N_PAD, d // 2), jnp.int32),
        scratch_types=[
            pltpu.VMEM((_ROWS_W, d // 2), jnp.int32),
            pltpu.VMEM((_SUP,), jnp.int32),
            pltpu.VMEM((_SUP + 2 * _L,), jnp.int32),
            pltpu.VMEM((_BLK, d // 2), jnp.int32),
            pltpu.VMEM((_BLK, d // 2), jnp.int32),
            pltpu.VMEM((_L,), jnp.int32),
            pltpu.SemaphoreType.DMA,
            pltpu.SemaphoreType.DMA,
        ],
    )
    def fold_kernel(hp_hbm, lsrc_hbm, ldst_hbm, cnt_hbm, out_hbm,
                    acc, sidx, didx, rows_a, rows_b, crow, sem0, sem1):
        cid = lax.axis_index("c")
        sid = lax.axis_index("s")
        wid = cid * 16 + sid
        lo = wid * _ROWS_W

        izero = jnp.zeros((_L,), jnp.int32)

        @pl.loop(0, _ROWS_W)
        def _(r):
            for c in range(d // (2 * _L)):
                acc[r, pl.ds(c * _L, _L)] = izero

        pltpu.sync_copy(cnt_hbm.at[wid], crow)
        total = crow[...][0]
        gmax = (total + _BLK - 1) // _BLK
        n_sup = (total + _SUP - 1) // _SUP

        def idx_ref(b):
            return sidx.at[pl.ds(b * _BLK, _BLK)]

        def issue(b, rows, sem):
            pltpu.async_copy(hp_hbm.at[idx_ref(b)], rows, sem)

        def fold_block(s, b, rows, sem):
            pltpu.make_async_copy(hp_hbm.at[idx_ref(b)], rows, sem).wait()
            gb = s * _BPS + b
            nb = jnp.minimum(total - gb * _BLK, _BLK)
            base = b * _BLK
            w = 2 * _L

            def fold_row(j, ld):
                ld_next = didx[pl.ds(base + j + 1, _L)][0]
                for c in range(0, d // w, 2):
                    sl0 = pl.ds(c * _L, _L)
                    sl1 = pl.ds((c + 1) * _L, _L)
                    a0 = plsc.bitcast(acc[ld, sl0], jnp.bfloat16)
                    r0 = plsc.bitcast(rows[j, sl0], jnp.bfloat16)
                    a1 = plsc.bitcast(acc[ld, sl1], jnp.bfloat16)
                    r1 = plsc.bitcast(rows[j, sl1], jnp.bfloat16)
                    acc[ld, sl0] = plsc.bitcast(jnp.maximum(a0, r0),
                                                jnp.int32)
                    acc[ld, sl1] = plsc.bitcast(jnp.maximum(a1, r1),
                                                jnp.int32)
                return ld_next

            ld0 = didx[pl.ds(base, _L)][0]

            @pl.when(nb == _BLK)
            def _():
                lax.fori_loop(0, _BLK, fold_row, ld0, unroll=2)

            @pl.when(nb < _BLK)
            def _():
                lax.fori_loop(0, nb, fold_row, ld0)

        @pl.loop(0, n_sup)
        def _(s):
            pltpu.sync_copy(lsrc_hbm.at[wid, pl.ds(s * _SUP, _SUP)], sidx)
            pltpu.sync_copy(ldst_hbm.at[wid, pl.ds(s * _SUP, _SUP)],
                            didx.at[pl.ds(0, _SUP)])
            nbh = jnp.minimum(gmax - s * _BPS, _BPS)

            @pl.when(nbh >= 1)
            def _():
                issue(0, rows_a, sem0)

            @pl.when(nbh >= 2)
            def _():
                issue(1, rows_b, sem1)

            @pl.loop(0, (nbh + 1) // 2)
            def _(t):
                b0 = 2 * t
                b1 = 2 * t + 1
                fold_block(s, b0, rows_a, sem0)

                @pl.when(b0 + 2 < nbh)
                def _():
                    issue(b0 + 2, rows_a, sem0)

                @pl.when(b1 < nbh)
                def _():
                    fold_block(s, b1, rows_b, sem1)

                    @pl.when(b1 + 2 < nbh)
                    def _():
                        issue(b1 + 2, rows_b, sem1)

        pltpu.sync_copy(acc, out_hbm.at[pl.ds(lo, _ROWS_W)])

    return fold_kernel(hp, lsrc, ldst, cnt)


def kernel(x, edge_index, Wp, bp, W1, W2):
    n, d = x.shape
    src = edge_index[0].astype(jnp.int32)
    dst = edge_index[1].astype(jnp.int32)
    e = src.shape[0]
    e_pad = ((e + _CHUNK - 1) // _CHUNK) * _CHUNK
    if e_pad != e:
        src = jnp.concatenate([src, jnp.zeros((e_pad - e,), jnp.int32)])
        dst = jnp.concatenate([dst, jnp.full((e_pad - e,), n, jnp.int32)])

    x_pad = jnp.zeros((_N_PAD, d), jnp.float32).at[:n].set(x)
    wpt = Wp.T
    bp8 = jnp.broadcast_to(bp.reshape(1, d), (8, d))
    w1t = W1.T
    w2t = W2.T
    k = d // 2
    wat1, wb1 = w1t[:d], w1t[d:]
    wat2, wb2 = w2t[:d], w2t[d:]
    wbl1, wbh1 = wb1[:k], wb1[k:]
    wbl2, wbh2 = wb2[:k], wb2[k:]

    lsrc, ldst, cnt = _edge_partition(src, dst)
    hp1 = _tc_pre(x_pad, wpt, bp8)
    ha1 = _fold(hp1, lsrc, ldst, cnt)
    h2, hp2 = _tc_post_hp(x_pad, ha1, wat1, wbl1, wbh1, wpt, bp8)
    ha2 = _fold(hp2, lsrc, ldst, cnt)
    h3 = _tc_post(h2, ha2, wat2, wbl2, wbh2)
    return h3[:n]


# speedup vs baseline: 4.1674x; 1.0052x over previous
import functools

import jax
import jax.numpy as jnp
from jax import lax
from jax.experimental import pallas as pl
from jax.experimental.pallas import tpu as pltpu
from jax.experimental.pallas import tpu_sc as plsc

_N_PAD = 10240
_NW = 32
_ROWS_W = _N_PAD // _NW
_CHUNK = 2000
_BLK = 128
_SUP = 4096
_BPS = _SUP // _BLK
_L = 16
_PCAP = _CHUNK + 2 * _BLK
_NBUF = 4

_SC_PARAMS = pltpu.CompilerParams(needs_layout_passes=False)


def _dot(a, b):
    return jnp.dot(a, b, preferred_element_type=jnp.float32,
                   precision=lax.Precision.HIGHEST)


def _pack_bf16(s):
    k = s.shape[1] // 2
    def bits(x):
        b = lax.bitcast_convert_type(x, jnp.int32)
        rnd = (lax.shift_right_logical(b, 16) & 1) + 0x7FFF
        return lax.shift_right_logical(b + rnd, 16)
    lo = bits(s[:, :k])
    hi = bits(s[:, k:])
    return lo | lax.shift_left(hi, 16)


def _unpack_bf16(w):
    lo = lax.bitcast_convert_type(lax.shift_left(w, 16), jnp.float32)
    hi = lax.bitcast_convert_type(w & jnp.int32(-65536), jnp.float32)
    return lo, hi


def _pre_body(h_ref, wpt_ref, bp_ref, o_ref):
    o_ref[...] = _pack_bf16(
        jax.nn.sigmoid(_dot(h_ref[...], wpt_ref[...]) + bp_ref[0:1, :]))


def _tc_pre(h, wpt, bp8):
    d = h.shape[1]
    rb = 1024
    return pl.pallas_call(
        _pre_body,
        grid=(_N_PAD // rb,),
        in_specs=[
            pl.BlockSpec((rb, d), lambda i: (i, 0)),
            pl.BlockSpec((d, d), lambda i: (0, 0)),
            pl.BlockSpec((8, d), lambda i: (0, 0)),
        ],
        out_specs=pl.BlockSpec((rb, d // 2), lambda i: (i, 0)),
        out_shape=jax.ShapeDtypeStruct((_N_PAD, d // 2), jnp.int32),
    )(h, wpt, bp8)


def _post_update(h, ha_words, wat, wbt_lo, wbt_hi):
    ha_lo, ha_hi = _unpack_bf16(ha_words)
    s = jax.nn.sigmoid(_dot(h, wat) + _dot(ha_lo, wbt_lo)
                       + _dot(ha_hi, wbt_hi))
    nrm = jnp.sqrt(jnp.sum(s * s, axis=1, keepdims=True))
    return s / jnp.maximum(nrm, 1e-12)


def _post_body(h_ref, ha_ref, wat_ref, wbl_ref, wbh_ref, o_ref):
    o_ref[...] = _post_update(h_ref[...], ha_ref[...], wat_ref[...],
                              wbl_ref[...], wbh_ref[...])


def _post_hp_body(h_ref, ha_ref, wat_ref, wbl_ref, wbh_ref, wpt_ref, bp_ref,
                  o_ref, ohp_ref):
    hn = _post_update(h_ref[...], ha_ref[...], wat_ref[...], wbl_ref[...],
                      wbh_ref[...])
    o_ref[...] = hn
    ohp_ref[...] = _pack_bf16(
        jax.nn.sigmoid(_dot(hn, wpt_ref[...]) + bp_ref[0:1, :]))


def _tc_post(h, ha, wat, wbt_lo, wbt_hi):
    d = h.shape[1]
    rb = 1024
    return pl.pallas_call(
        _post_body,
        grid=(_N_PAD // rb,),
        in_specs=[
            pl.BlockSpec((rb, d), lambda i: (i, 0)),
            pl.BlockSpec((rb, d // 2), lambda i: (i, 0)),
            pl.BlockSpec((d, d), lambda i: (0, 0)),
            pl.BlockSpec((d // 2, d), lambda i: (0, 0)),
            pl.BlockSpec((d // 2, d), lambda i: (0, 0)),
        ],
        out_specs=pl.BlockSpec((rb, d), lambda i: (i, 0)),
        out_shape=jax.ShapeDtypeStruct((_N_PAD, d), jnp.float32),
    )(h, ha, wat, wbt_lo, wbt_hi)


def _tc_post_hp(h, ha, wat, wbt_lo, wbt_hi, wpt, bp8):
    d = h.shape[1]
    rb = 1024
    return pl.pallas_call(
        _post_hp_body,
        grid=(_N_PAD // rb,),
        in_specs=[
            pl.BlockSpec((rb, d), lambda i: (i, 0)),
            pl.BlockSpec((rb, d // 2), lambda i: (i, 0)),
            pl.BlockSpec((d, d), lambda i: (0, 0)),
            pl.BlockSpec((d // 2, d), lambda i: (0, 0)),
            pl.BlockSpec((d // 2, d), lambda i: (0, 0)),
            pl.BlockSpec((d, d), lambda i: (0, 0)),
            pl.BlockSpec((8, d), lambda i: (0, 0)),
        ],
        out_specs=[
            pl.BlockSpec((rb, d), lambda i: (i, 0)),
            pl.BlockSpec((rb, d // 2), lambda i: (i, 0)),
        ],
        out_shape=[
            jax.ShapeDtypeStruct((_N_PAD, d), jnp.float32),
            jax.ShapeDtypeStruct((_N_PAD, d // 2), jnp.int32),
        ],
    )(h, ha, wat, wbt_lo, wbt_hi, wpt, bp8)


def _edge_partition(src, dst):
    e_pad = src.shape[0]
    n_chunks = e_pad // _CHUNK
    n_pairs = (n_chunks + 1) // 2
    cap = ((e_pad + _SUP - 1) // _SUP) * _SUP
    mesh = plsc.VectorSubcoreMesh(core_axis_name="c", subcore_axis_name="s")

    @functools.partial(
        pl.kernel,
        mesh=mesh,
        compiler_params=_SC_PARAMS,
        out_type=[
            jax.ShapeDtypeStruct((_NW, cap), jnp.int32),
            jax.ShapeDtypeStruct((_NW, cap), jnp.int32),
            jax.ShapeDtypeStruct((_NW, _L), jnp.int32),
        ],
        scratch_types=[
            pltpu.VMEM((_CHUNK,), jnp.int32),
            pltpu.VMEM((_CHUNK,), jnp.int32),
            pltpu.VMEM((_CHUNK,), jnp.int32),
            pltpu.VMEM((_CHUNK,), jnp.int32),
            pltpu.VMEM((_PCAP,), jnp.int32),
            pltpu.VMEM((_PCAP,), jnp.int32),
            pltpu.SMEM((2,), jnp.int32),
            pltpu.VMEM((_L,), jnp.int32),
            pltpu.SemaphoreType.DMA,
            pltpu.SemaphoreType.DMA,
        ],
    )
    def part_kernel(src_hbm, dst_hbm, lsrc_hbm, ldst_hbm, cnt_hbm,
                    src_a, src_b, dst_a, dst_b, psrc, pdst, st, crow,
                    sem0, sem1):
        cid = lax.axis_index("c")
        sid = lax.axis_index("s")
        wid = cid * 16 + sid
        lo = wid * _ROWS_W
        hi = lo + _ROWS_W

        izero = jnp.zeros((_L,), jnp.int32)

        @pl.loop(0, _PCAP, step=_L)
        def _(i):
            psrc[pl.ds(i, _L)] = izero

        st[0] = 0
        st[1] = 0

        def stage(cix, sbuf, dbuf, sem):
            base = cix * _CHUNK
            pltpu.async_copy(src_hbm.at[pl.ds(base, _CHUNK)], sbuf, sem)
            pltpu.async_copy(dst_hbm.at[pl.ds(base, _CHUNK)], dbuf, sem)

        def wait(cix, sbuf, dbuf, sem):
            pltpu.make_async_copy(src_hbm.at[pl.ds(cix * _CHUNK, _CHUNK)],
                                  sbuf, sem).wait()
            pltpu.make_async_copy(dst_hbm.at[pl.ds(cix * _CHUNK, _CHUNK)],
                                  dbuf, sem).wait()

        def scan(sbuf, dbuf):
            kv = jnp.full((_L,), st[0] - 1, jnp.int32)

            def group(g, kv):
                i = g * _L
                dvec = dbuf[pl.ds(i, _L)]
                svec = sbuf[pl.ds(i, _L)]
                m = (dvec >= lo) & (dvec < hi)
                pos = kv + jnp.cumsum(m.astype(jnp.int32))
                plsc.store_scatter(psrc, [pos], svec, mask=m)
                plsc.store_scatter(pdst, [pos], dvec - lo, mask=m)
                return kv + plsc.all_reduce_population_count(m)

            kv = lax.fori_loop(0, _CHUNK // _L, group, kv, unroll=4)
            k = kv[0] + 1
            nb = k // _BLK

            def emit(b, carry):
                bo = st[1] + b
                pltpu.sync_copy(
                    psrc.at[pl.ds(b * _BLK, _BLK)],
                    lsrc_hbm.at[wid, pl.ds(bo * _BLK, _BLK)])
                pltpu.sync_copy(
                    pdst.at[pl.ds(b * _BLK, _BLK)],
                    ldst_hbm.at[wid, pl.ds(bo * _BLK, _BLK)])
                return carry

            lax.fori_loop(0, nb, emit, 0)
            st[1] = st[1] + nb
            base = nb * _BLK

            @pl.when(nb > 0)
            def _():
                for c in range(_BLK // _L):
                    off = c * _L
                    psrc[pl.ds(off, _L)] = psrc[pl.ds(base + off, _L)]
                    pdst[pl.ds(off, _L)] = pdst[pl.ds(base + off, _L)]

            st[0] = k - base

        stage(0, src_a, dst_a, sem0)
        stage(1, src_b, dst_b, sem1)

        @pl.loop(0, n_pairs)
        def _(t):
            c0 = 2 * t
            c1 = 2 * t + 1
            wait(c0, src_a, dst_a, sem0)
            scan(src_a, dst_a)

            @pl.when(c0 + 2 < n_chunks)
            def _():
                stage(c0 + 2, src_a, dst_a, sem0)

            @pl.when(c1 < n_chunks)
            def _():
                wait(c1, src_b, dst_b, sem1)
                scan(src_b, dst_b)

                @pl.when(c1 + 2 < n_chunks)
                def _():
                    stage(c1 + 2, src_b, dst_b, sem1)

        total = st[1] * _BLK + st[0]

        @pl.when(st[0] > 0)
        def _():
            b = st[1]
            pltpu.sync_copy(psrc.at[pl.ds(0, _BLK)],
                            lsrc_hbm.at[wid, pl.ds(b * _BLK, _BLK)])
            pltpu.sync_copy(pdst.at[pl.ds(0, _BLK)],
                            ldst_hbm.at[wid, pl.ds(b * _BLK, _BLK)])

        crow[...] = jnp.full((_L,), total, jnp.int32)
        pltpu.sync_copy(crow, cnt_hbm.at[wid])

    return part_kernel(src, dst)


def _fold(hp, lsrc, ldst, cnt):
    d = 2 * hp.shape[1]
    mesh = plsc.VectorSubcoreMesh(core_axis_name="c", subcore_axis_name="s")

    @functools.partial(
        pl.kernel,
        mesh=mesh,
        compiler_params=_SC_PARAMS,
        out_type=jax.ShapeDtypeStruct((_N_PAD, d // 2), jnp.int32),
        scratch_types=[
            pltpu.VMEM((_ROWS_W, d // 2), jnp.int32),
            pltpu.VMEM((_SUP,), jnp.int32),
            pltpu.VMEM((_SUP + 2 * _L,), jnp.int32),
            pltpu.VMEM((_BLK, d // 2), jnp.int32),
            pltpu.VMEM((_BLK, d // 2), jnp.int32),
            pltpu.VMEM((_BLK, d // 2), jnp.int32),
            pltpu.VMEM((_BLK, d // 2), jnp.int32),
            pltpu.VMEM((_L,), jnp.int32),
            pltpu.SemaphoreType.DMA,
            pltpu.SemaphoreType.DMA,
            pltpu.SemaphoreType.DMA,
            pltpu.SemaphoreType.DMA,
        ],
    )
    def fold_kernel(hp_hbm, lsrc_hbm, ldst_hbm, cnt_hbm, out_hbm,
                    acc, sidx, didx, rows_a, rows_b, rows_c, rows_d,
                    crow, sem0, sem1, sem2, sem3):
        cid = lax.axis_index("c")
        sid = lax.axis_index("s")
        wid = cid * 16 + sid
        lo = wid * _ROWS_W

        izero = jnp.zeros((_L,), jnp.int32)

        @pl.loop(0, _ROWS_W)
        def _(r):
            for c in range(d // (2 * _L)):
                acc[r, pl.ds(c * _L, _L)] = izero

        pltpu.sync_copy(cnt_hbm.at[wid], crow)
        total = crow[...][0]
        gmax = (total + _BLK - 1) // _BLK
        n_sup = (total + _SUP - 1) // _SUP

        def idx_ref(b):
            return sidx.at[pl.ds(b * _BLK, _BLK)]

        def issue(b, rows, sem):
            pltpu.async_copy(hp_hbm.at[idx_ref(b)], rows, sem)

        def fold_block(s, b, rows, sem, nbh):
            pltpu.make_async_copy(hp_hbm.at[idx_ref(b)], rows, sem).wait()
            gb = s * _BPS + b
            nb = jnp.minimum(total - gb * _BLK, _BLK)
            base = b * _BLK
            w = 2 * _L

            def fold_row(j, ld):
                ld_next = didx[pl.ds(base + j + 1, _L)][0]
                for c in range(0, d // w, 2):
                    sl0 = pl.ds(c * _L, _L)
                    sl1 = pl.ds((c + 1) * _L, _L)
                    a0 = plsc.bitcast(acc[ld, sl0], jnp.bfloat16)
                    r0 = plsc.bitcast(rows[j, sl0], jnp.bfloat16)
                    a1 = plsc.bitcast(acc[ld, sl1], jnp.bfloat16)
                    r1 = plsc.bitcast(rows[j, sl1], jnp.bfloat16)
                    acc[ld, sl0] = plsc.bitcast(jnp.maximum(a0, r0),
                                                jnp.int32)
                    acc[ld, sl1] = plsc.bitcast(jnp.maximum(a1, r1),
                                                jnp.int32)
                return ld_next

            ld0 = didx[pl.ds(base, _L)][0]

            @pl.when(nb == _BLK)
            def _():
                lax.fori_loop(0, _BLK, fold_row, ld0, unroll=2)

            @pl.when(nb < _BLK)
            def _():
                lax.fori_loop(0, nb, fold_row, ld0)

        @pl.loop(0, n_sup)
        def _(s):
            pltpu.sync_copy(lsrc_hbm.at[wid, pl.ds(s * _SUP, _SUP)], sidx)
            pltpu.sync_copy(ldst_hbm.at[wid, pl.ds(s * _SUP, _SUP)],
                            didx.at[pl.ds(0, _SUP)])
            nbh = jnp.minimum(gmax - s * _BPS, _BPS)
            bufs = ((rows_a, sem0), (rows_b, sem1), (rows_c, sem2),
                    (rows_d, sem3))

            for i, (rows, sem) in enumerate(bufs):
                @pl.when(nbh >= i + 1)
                def _(i=i, rows=rows, sem=sem):
                    issue(i, rows, sem)

            @pl.loop(0, (nbh + _NBUF - 1) // _NBUF)
            def _(t):
                for i, (rows, sem) in enumerate(bufs):
                    b = _NBUF * t + i

                    @pl.when(b < nbh)
                    def _(b=b, rows=rows, sem=sem):
                        fold_block(s, b, rows, sem, nbh)

                        @pl.when(b + _NBUF < nbh)
                        def _():
                            pltpu.async_copy(hp_hbm.at[idx_ref(b + _NBUF)],
                                             rows, sem)

        pltpu.sync_copy(acc, out_hbm.at[pl.ds(lo, _ROWS_W)])

    return fold_kernel(hp, lsrc, ldst, cnt)


def kernel(x, edge_index, Wp, bp, W1, W2):
    n, d = x.shape
    src = edge_index[0].astype(jnp.int32)
    dst = edge_index[1].astype(jnp.int32)
    e = src.shape[0]
    e_pad = ((e + _CHUNK - 1) // _CHUNK) * _CHUNK
    if e_pad != e:
        src = jnp.concatenate([src, jnp.zeros((e_pad - e,), jnp.int32)])
        dst = jnp.concatenate([dst, jnp.full((e_pad - e,), n, jnp.int32)])

    x_pad = jnp.zeros((_N_PAD, d), jnp.float32).at[:n].set(x)
    wpt = Wp.T
    bp8 = jnp.broadcast_to(bp.reshape(1, d), (8, d))
    w1t = W1.T
    w2t = W2.T
    k = d // 2
    wat1, wb1 = w1t[:d], w1t[d:]
    wat2, wb2 = w2t[:d], w2t[d:]
    wbl1, wbh1 = wb1[:k], wb1[k:]
    wbl2, wbh2 = wb2[:k], wb2[k:]

    lsrc, ldst, cnt = _edge_partition(src, dst)
    hp1 = _tc_pre(x_pad, wpt, bp8)
    ha1 = _fold(hp1, lsrc, ldst, cnt)
    h2, hp2 = _tc_post_hp(x_pad, ha1, wat1, wbl1, wbh1, wpt, bp8)
    ha2 = _fold(hp2, lsrc, ldst, cnt)
    h3 = _tc_post(h2, ha2, wat2, wbl2, wbh2)
    return h3[:n]


# speedup vs baseline: 4.1756x; 1.0020x over previous
import functools

import jax
import jax.numpy as jnp
from jax import lax
from jax.experimental import pallas as pl
from jax.experimental.pallas import tpu as pltpu
from jax.experimental.pallas import tpu_sc as plsc

_N_PAD = 10240
_NW = 32
_ROWS_W = _N_PAD // _NW
_CHUNK = 2000
_BLK = 128
_SUP = 4096
_BPS = _SUP // _BLK
_L = 16
_PCAP = _CHUNK + 2 * _BLK
_NBUF = 4

_SC_PARAMS = pltpu.CompilerParams(needs_layout_passes=False)


def _dot(a, b):
    return jnp.dot(a, b, preferred_element_type=jnp.float32,
                   precision=lax.Precision.HIGHEST)


def _pack_bf16(s):
    k = s.shape[1] // 2
    def bits(x):
        b = lax.bitcast_convert_type(x, jnp.int32)
        rnd = (lax.shift_right_logical(b, 16) & 1) + 0x7FFF
        return lax.shift_right_logical(b + rnd, 16)
    lo = bits(s[:, :k])
    hi = bits(s[:, k:])
    return lo | lax.shift_left(hi, 16)


def _unpack_bf16(w):
    lo = lax.bitcast_convert_type(lax.shift_left(w, 16), jnp.float32)
    hi = lax.bitcast_convert_type(w & jnp.int32(-65536), jnp.float32)
    return lo, hi


def _pre_body(h_ref, wpt_ref, bp_ref, o_ref):
    o_ref[...] = _pack_bf16(
        jax.nn.sigmoid(_dot(h_ref[...], wpt_ref[...]) + bp_ref[0:1, :]))


def _tc_pre(h, wpt, bp8):
    d = h.shape[1]
    rb = 1024
    return pl.pallas_call(
        _pre_body,
        grid=(_N_PAD // rb,),
        in_specs=[
            pl.BlockSpec((rb, d), lambda i: (i, 0)),
            pl.BlockSpec((d, d), lambda i: (0, 0)),
            pl.BlockSpec((8, d), lambda i: (0, 0)),
        ],
        out_specs=pl.BlockSpec((rb, d // 2), lambda i: (i, 0)),
        out_shape=jax.ShapeDtypeStruct((_N_PAD, d // 2), jnp.int32),
    )(h, wpt, bp8)


def _post_update(h, ha_words, wat, wbt_lo, wbt_hi):
    ha_lo, ha_hi = _unpack_bf16(ha_words)
    s = jax.nn.sigmoid(_dot(h, wat) + _dot(ha_lo, wbt_lo)
                       + _dot(ha_hi, wbt_hi))
    nrm = jnp.sqrt(jnp.sum(s * s, axis=1, keepdims=True))
    return s / jnp.maximum(nrm, 1e-12)


def _post_body(h_ref, ha_ref, wat_ref, wbl_ref, wbh_ref, o_ref):
    o_ref[...] = _post_update(h_ref[...], ha_ref[...], wat_ref[...],
                              wbl_ref[...], wbh_ref[...])


def _post_hp_body(h_ref, ha_ref, wat_ref, wbl_ref, wbh_ref, wpt_ref, bp_ref,
                  o_ref, ohp_ref):
    hn = _post_update(h_ref[...], ha_ref[...], wat_ref[...], wbl_ref[...],
                      wbh_ref[...])
    o_ref[...] = hn
    ohp_ref[...] = _pack_bf16(
        jax.nn.sigmoid(_dot(hn, wpt_ref[...]) + bp_ref[0:1, :]))


def _tc_post(h, ha, wat, wbt_lo, wbt_hi):
    d = h.shape[1]
    rb = 1024
    return pl.pallas_call(
        _post_body,
        grid=(_N_PAD // rb,),
        in_specs=[
            pl.BlockSpec((rb, d), lambda i: (i, 0)),
            pl.BlockSpec((rb, d // 2), lambda i: (i, 0)),
            pl.BlockSpec((d, d), lambda i: (0, 0)),
            pl.BlockSpec((d // 2, d), lambda i: (0, 0)),
            pl.BlockSpec((d // 2, d), lambda i: (0, 0)),
        ],
        out_specs=pl.BlockSpec((rb, d), lambda i: (i, 0)),
        out_shape=jax.ShapeDtypeStruct((_N_PAD, d), jnp.float32),
    )(h, ha, wat, wbt_lo, wbt_hi)


def _tc_post_hp(h, ha, wat, wbt_lo, wbt_hi, wpt, bp8):
    d = h.shape[1]
    rb = 1024
    return pl.pallas_call(
        _post_hp_body,
        grid=(_N_PAD // rb,),
        in_specs=[
            pl.BlockSpec((rb, d), lambda i: (i, 0)),
            pl.BlockSpec((rb, d // 2), lambda i: (i, 0)),
            pl.BlockSpec((d, d), lambda i: (0, 0)),
            pl.BlockSpec((d // 2, d), lambda i: (0, 0)),
            pl.BlockSpec((d // 2, d), lambda i: (0, 0)),
            pl.BlockSpec((d, d), lambda i: (0, 0)),
            pl.BlockSpec((8, d), lambda i: (0, 0)),
        ],
        out_specs=[
            pl.BlockSpec((rb, d), lambda i: (i, 0)),
            pl.BlockSpec((rb, d // 2), lambda i: (i, 0)),
        ],
        out_shape=[
            jax.ShapeDtypeStruct((_N_PAD, d), jnp.float32),
            jax.ShapeDtypeStruct((_N_PAD, d // 2), jnp.int32),
        ],
    )(h, ha, wat, wbt_lo, wbt_hi, wpt, bp8)


def _edge_partition(src, dst):
    e_pad = src.shape[0]
    n_chunks = e_pad // _CHUNK
    n_pairs = (n_chunks + 1) // 2
    cap = ((e_pad + _SUP - 1) // _SUP) * _SUP
    mesh = plsc.VectorSubcoreMesh(core_axis_name="c", subcore_axis_name="s")

    @functools.partial(
        pl.kernel,
        mesh=mesh,
        compiler_params=_SC_PARAMS,
        out_type=[
            jax.ShapeDtypeStruct((_NW, cap), jnp.int32),
            jax.ShapeDtypeStruct((_NW, cap), jnp.int32),
            jax.ShapeDtypeStruct((_NW, _L), jnp.int32),
        ],
        scratch_types=[
            pltpu.VMEM((_CHUNK,), jnp.int32),
            pltpu.VMEM((_CHUNK,), jnp.int32),
            pltpu.VMEM((_CHUNK,), jnp.int32),
            pltpu.VMEM((_CHUNK,), jnp.int32),
            pltpu.VMEM((_PCAP,), jnp.int32),
            pltpu.VMEM((_PCAP,), jnp.int32),
            pltpu.SMEM((2,), jnp.int32),
            pltpu.VMEM((_L,), jnp.int32),
            pltpu.VMEM_SHARED((e_pad,), jnp.int32),
            pltpu.VMEM_SHARED((e_pad,), jnp.int32),
            pltpu.SemaphoreType.DMA,
            pltpu.SemaphoreType.DMA,
        ],
    )
    def part_kernel(src_hbm, dst_hbm, lsrc_hbm, ldst_hbm, cnt_hbm,
                    src_a, src_b, dst_a, dst_b, psrc, pdst, st, crow,
                    src_sh, dst_sh, sem0, sem1):
        cid = lax.axis_index("c")
        sid = lax.axis_index("s")
        wid = cid * 16 + sid
        lo = wid * _ROWS_W
        hi = lo + _ROWS_W

        izero = jnp.zeros((_L,), jnp.int32)

        @pl.loop(0, _PCAP, step=_L)
        def _(i):
            psrc[pl.ds(i, _L)] = izero

        st[0] = 0
        st[1] = 0

        @pl.when(sid == 0)
        def _():
            pltpu.sync_copy(src_hbm, src_sh)
            pltpu.sync_copy(dst_hbm, dst_sh)

        plsc.subcore_barrier()

        def stage(cix, sbuf, dbuf, sem):
            base = cix * _CHUNK
            pltpu.async_copy(src_sh.at[pl.ds(base, _CHUNK)], sbuf, sem)
            pltpu.async_copy(dst_sh.at[pl.ds(base, _CHUNK)], dbuf, sem)

        def wait(cix, sbuf, dbuf, sem):
            pltpu.make_async_copy(src_sh.at[pl.ds(cix * _CHUNK, _CHUNK)],
                                  sbuf, sem).wait()
            pltpu.make_async_copy(dst_sh.at[pl.ds(cix * _CHUNK, _CHUNK)],
                                  dbuf, sem).wait()

        def scan(sbuf, dbuf):
            kv = jnp.full((_L,), st[0] - 1, jnp.int32)

            def group(g, kv):
                i = g * _L
                dvec = dbuf[pl.ds(i, _L)]
                svec = sbuf[pl.ds(i, _L)]
                m = (dvec >= lo) & (dvec < hi)
                pos = kv + jnp.cumsum(m.astype(jnp.int32))
                plsc.store_scatter(psrc, [pos], svec, mask=m)
                plsc.store_scatter(pdst, [pos], dvec - lo, mask=m)
                return kv + plsc.all_reduce_population_count(m)

            kv = lax.fori_loop(0, _CHUNK // _L, group, kv, unroll=4)
            k = kv[0] + 1
            nb = k // _BLK

            def emit(b, carry):
                bo = st[1] + b
                pltpu.sync_copy(
                    psrc.at[pl.ds(b * _BLK, _BLK)],
                    lsrc_hbm.at[wid, pl.ds(bo * _BLK, _BLK)])
                pltpu.sync_copy(
                    pdst.at[pl.ds(b * _BLK, _BLK)],
                    ldst_hbm.at[wid, pl.ds(bo * _BLK, _BLK)])
                return carry

            lax.fori_loop(0, nb, emit, 0)
            st[1] = st[1] + nb
            base = nb * _BLK

            @pl.when(nb > 0)
            def _():
                for c in range(_BLK // _L):
                    off = c * _L
                    psrc[pl.ds(off, _L)] = psrc[pl.ds(base + off, _L)]
                    pdst[pl.ds(off, _L)] = pdst[pl.ds(base + off, _L)]

            st[0] = k - base

        stage(0, src_a, dst_a, sem0)
        stage(1, src_b, dst_b, sem1)

        @pl.loop(0, n_pairs)
        def _(t):
            c0 = 2 * t
            c1 = 2 * t + 1
            wait(c0, src_a, dst_a, sem0)
            scan(src_a, dst_a)

            @pl.when(c0 + 2 < n_chunks)
            def _():
                stage(c0 + 2, src_a, dst_a, sem0)

            @pl.when(c1 < n_chunks)
            def _():
                wait(c1, src_b, dst_b, sem1)
                scan(src_b, dst_b)

                @pl.when(c1 + 2 < n_chunks)
                def _():
                    stage(c1 + 2, src_b, dst_b, sem1)

        total = st[1] * _BLK + st[0]

        @pl.when(st[0] > 0)
        def _():
            b = st[1]
            pltpu.sync_copy(psrc.at[pl.ds(0, _BLK)],
                            lsrc_hbm.at[wid, pl.ds(b * _BLK, _BLK)])
            pltpu.sync_copy(pdst.at[pl.ds(0, _BLK)],
                            ldst_hbm.at[wid, pl.ds(b * _BLK, _BLK)])

        crow[...] = jnp.full((_L,), total, jnp.int32)
        pltpu.sync_copy(crow, cnt_hbm.at[wid])

    return part_kernel(src, dst)


def _fold(hp, lsrc, ldst, cnt):
    d = 2 * hp.shape[1]
    mesh = plsc.VectorSubcoreMesh(core_axis_name="c", subcore_axis_name="s")

    @functools.partial(
        pl.kernel,
        mesh=mesh,
        compiler_params=_SC_PARAMS,
        out_type=jax.ShapeDtypeStruct((_N_PAD, d // 2), jnp.int32),
        scratch_types=[
            pltpu.VMEM((_ROWS_W, d // 2), jnp.int32),
            pltpu.VMEM((_SUP,), jnp.int32),
            pltpu.VMEM((_SUP + 2 * _L,), jnp.int32),
            pltpu.VMEM((_BLK, d // 2), jnp.int32),
            pltpu.VMEM((_BLK, d // 2), jnp.int32),
            pltpu.VMEM((_BLK, d // 2), jnp.int32),
            pltpu.VMEM((_BLK, d // 2), jnp.int32),
            pltpu.VMEM((_L,), jnp.int32),
            pltpu.SemaphoreType.DMA,
            pltpu.SemaphoreType.DMA,
            pltpu.SemaphoreType.DMA,
            pltpu.SemaphoreType.DMA,
        ],
    )
    def fold_kernel(hp_hbm, lsrc_hbm, ldst_hbm, cnt_hbm, out_hbm,
                    acc, sidx, didx, rows_a, rows_b, rows_c, rows_d,
                    crow, sem0, sem1, sem2, sem3):
        cid = lax.axis_index("c")
        sid = lax.axis_index("s")
        wid = cid * 16 + sid
        lo = wid * _ROWS_W

        izero = jnp.zeros((_L,), jnp.int32)

        @pl.loop(0, _ROWS_W)
        def _(r):
            for c in range(d // (2 * _L)):
                acc[r, pl.ds(c * _L, _L)] = izero

        pltpu.sync_copy(cnt_hbm.at[wid], crow)
        total = crow[...][0]
        gmax = (total + _BLK - 1) // _BLK
        n_sup = (total + _SUP - 1) // _SUP

        def idx_ref(b):
            return sidx.at[pl.ds(b * _BLK, _BLK)]

        def issue(b, rows, sem):
            pltpu.async_copy(hp_hbm.at[idx_ref(b)], rows, sem)

        def fold_block(s, b, rows, sem, nbh):
            pltpu.make_async_copy(hp_hbm.at[idx_ref(b)], rows, sem).wait()
            gb = s * _BPS + b
            nb = jnp.minimum(total - gb * _BLK, _BLK)
            base = b * _BLK
            w = 2 * _L

            def fold_row(j, ld):
                ld_next = didx[pl.ds(base + j + 1, _L)][0]
                for c in range(0, d // w, 2):
                    sl0 = pl.ds(c * _L, _L)
                    sl1 = pl.ds((c + 1) * _L, _L)
                    a0 = plsc.bitcast(acc[ld, sl0], jnp.bfloat16)
                    r0 = plsc.bitcast(rows[j, sl0], jnp.bfloat16)
                    a1 = plsc.bitcast(acc[ld, sl1], jnp.bfloat16)
                    r1 = plsc.bitcast(rows[j, sl1], jnp.bfloat16)
                    acc[ld, sl0] = plsc.bitcast(jnp.maximum(a0, r0),
                                                jnp.int32)
                    acc[ld, sl1] = plsc.bitcast(jnp.maximum(a1, r1),
                                                jnp.int32)
                return ld_next

            ld0 = didx[pl.ds(base, _L)][0]

            @pl.when(nb == _BLK)
            def _():
                lax.fori_loop(0, _BLK, fold_row, ld0, unroll=2)

            @pl.when(nb < _BLK)
            def _():
                lax.fori_loop(0, nb, fold_row, ld0)

        @pl.loop(0, n_sup)
        def _(s):
            pltpu.sync_copy(lsrc_hbm.at[wid, pl.ds(s * _SUP, _SUP)], sidx)
            pltpu.sync_copy(ldst_hbm.at[wid, pl.ds(s * _SUP, _SUP)],
                            didx.at[pl.ds(0, _SUP)])
            nbh = jnp.minimum(gmax - s * _BPS, _BPS)
            bufs = ((rows_a, sem0), (rows_b, sem1), (rows_c, sem2),
                    (rows_d, sem3))

            for i, (rows, sem) in enumerate(bufs):
                @pl.when(nbh >= i + 1)
                def _(i=i, rows=rows, sem=sem):
                    issue(i, rows, sem)

            @pl.loop(0, (nbh + _NBUF - 1) // _NBUF)
            def _(t):
                for i, (rows, sem) in enumerate(bufs):
                    b = _NBUF * t + i

                    @pl.when(b < nbh)
                    def _(b=b, rows=rows, sem=sem):
                        fold_block(s, b, rows, sem, nbh)

                        @pl.when(b + _NBUF < nbh)
                        def _():
                            pltpu.async_copy(hp_hbm.at[idx_ref(b + _NBUF)],
                                             rows, sem)

        pltpu.sync_copy(acc, out_hbm.at[pl.ds(lo, _ROWS_W)])

    return fold_kernel(hp, lsrc, ldst, cnt)


def kernel(x, edge_index, Wp, bp, W1, W2):
    n, d = x.shape
    src = edge_index[0].astype(jnp.int32)
    dst = edge_index[1].astype(jnp.int32)
    e = src.shape[0]
    e_pad = ((e + _CHUNK - 1) // _CHUNK) * _CHUNK
    if e_pad != e:
        src = jnp.concatenate([src, jnp.zeros((e_pad - e,), jnp.int32)])
        dst = jnp.concatenate([dst, jnp.full((e_pad - e,), n, jnp.int32)])

    x_pad = jnp.zeros((_N_PAD, d), jnp.float32).at[:n].set(x)
    wpt = Wp.T
    bp8 = jnp.broadcast_to(bp.reshape(1, d), (8, d))
    w1t = W1.T
    w2t = W2.T
    k = d // 2
    wat1, wb1 = w1t[:d], w1t[d:]
    wat2, wb2 = w2t[:d], w2t[d:]
    wbl1, wbh1 = wb1[:k], wb1[k:]
    wbl2, wbh2 = wb2[:k], wb2[k:]

    lsrc, ldst, cnt = _edge_partition(src, dst)
    hp1 = _tc_pre(x_pad, wpt, bp8)
    ha1 = _fold(hp1, lsrc, ldst, cnt)
    h2, hp2 = _tc_post_hp(x_pad, ha1, wat1, wbl1, wbh1, wpt, bp8)
    ha2 = _fold(hp2, lsrc, ldst, cnt)
    h3 = _tc_post(h2, ha2, wat2, wbl2, wbh2)
    return h3[:n]


# speedup vs baseline: 5.0504x; 1.2095x over previous
import functools

import jax
import jax.numpy as jnp
from jax import lax
from jax.experimental import pallas as pl
from jax.experimental.pallas import tpu as pltpu
from jax.experimental.pallas import tpu_sc as plsc

_N_PAD = 10240
_NW = 32
_ROWS_W = _N_PAD // _NW
_CHUNK = 4000
_BLK = 128
_SUP = 4096
_BPS = _SUP // _BLK
_L = 16
_PCAP = _CHUNK + 2 * _BLK
_NBUF = 4

_SC_PARAMS = pltpu.CompilerParams(needs_layout_passes=False)


def _dot(a, b):
    return jnp.dot(a, b, preferred_element_type=jnp.float32,
                   precision=lax.Precision.HIGHEST)


def _pack_bf16(s):
    k = s.shape[1] // 2
    def bits(x):
        b = lax.bitcast_convert_type(x, jnp.int32)
        rnd = (lax.shift_right_logical(b, 16) & 1) + 0x7FFF
        return lax.shift_right_logical(b + rnd, 16)
    lo = bits(s[:, :k])
    hi = bits(s[:, k:])
    return lo | lax.shift_left(hi, 16)


def _unpack_bf16(w):
    lo = lax.bitcast_convert_type(lax.shift_left(w, 16), jnp.float32)
    hi = lax.bitcast_convert_type(w & jnp.int32(-65536), jnp.float32)
    return lo, hi


def _dot_fast(a, b):
    return jnp.dot(a.astype(jnp.bfloat16), b.astype(jnp.bfloat16),
                   preferred_element_type=jnp.float32)


def _pre_body(h_ref, wpt_ref, bp_ref, o_ref):
    o_ref[...] = _pack_bf16(
        jax.nn.sigmoid(_dot_fast(h_ref[...], wpt_ref[...]) + bp_ref[0:1, :]))


def _tc_pre(h, wpt, bp8):
    d = h.shape[1]
    rb = 1024
    return pl.pallas_call(
        _pre_body,
        grid=(_N_PAD // rb,),
        in_specs=[
            pl.BlockSpec((rb, d), lambda i: (i, 0)),
            pl.BlockSpec((d, d), lambda i: (0, 0)),
            pl.BlockSpec((8, d), lambda i: (0, 0)),
        ],
        out_specs=pl.BlockSpec((rb, d // 2), lambda i: (i, 0)),
        out_shape=jax.ShapeDtypeStruct((_N_PAD, d // 2), jnp.int32),
    )(h, wpt, bp8)


def _post_update(h, ha_words, wat, wbt_lo, wbt_hi):
    ha_lo, ha_hi = _unpack_bf16(ha_words)
    s = jax.nn.sigmoid(_dot(h, wat) + _dot(ha_lo, wbt_lo)
                       + _dot(ha_hi, wbt_hi))
    nrm = jnp.sqrt(jnp.sum(s * s, axis=1, keepdims=True))
    return s / jnp.maximum(nrm, 1e-12)


def _post_body(h_ref, ha_ref, wat_ref, wbl_ref, wbh_ref, o_ref):
    o_ref[...] = _post_update(h_ref[...], ha_ref[...], wat_ref[...],
                              wbl_ref[...], wbh_ref[...])


def _post_hp_body(h_ref, ha_ref, wat_ref, wbl_ref, wbh_ref, wpt_ref, bp_ref,
                  o_ref, ohp_ref):
    hn = _post_update(h_ref[...], ha_ref[...], wat_ref[...], wbl_ref[...],
                      wbh_ref[...])
    o_ref[...] = hn
    ohp_ref[...] = _pack_bf16(
        jax.nn.sigmoid(_dot_fast(hn, wpt_ref[...]) + bp_ref[0:1, :]))


def _tc_post(h, ha, wat, wbt_lo, wbt_hi):
    d = h.shape[1]
    rb = 1024
    return pl.pallas_call(
        _post_body,
        grid=(_N_PAD // rb,),
        in_specs=[
            pl.BlockSpec((rb, d), lambda i: (i, 0)),
            pl.BlockSpec((rb, d // 2), lambda i: (i, 0)),
            pl.BlockSpec((d, d), lambda i: (0, 0)),
            pl.BlockSpec((d // 2, d), lambda i: (0, 0)),
            pl.BlockSpec((d // 2, d), lambda i: (0, 0)),
        ],
        out_specs=pl.BlockSpec((rb, d), lambda i: (i, 0)),
        out_shape=jax.ShapeDtypeStruct((_N_PAD, d), jnp.float32),
    )(h, ha, wat, wbt_lo, wbt_hi)


def _tc_post_hp(h, ha, wat, wbt_lo, wbt_hi, wpt, bp8):
    d = h.shape[1]
    rb = 1024
    return pl.pallas_call(
        _post_hp_body,
        grid=(_N_PAD // rb,),
        in_specs=[
            pl.BlockSpec((rb, d), lambda i: (i, 0)),
            pl.BlockSpec((rb, d // 2), lambda i: (i, 0)),
            pl.BlockSpec((d, d), lambda i: (0, 0)),
            pl.BlockSpec((d // 2, d), lambda i: (0, 0)),
            pl.BlockSpec((d // 2, d), lambda i: (0, 0)),
            pl.BlockSpec((d, d), lambda i: (0, 0)),
            pl.BlockSpec((8, d), lambda i: (0, 0)),
        ],
        out_specs=[
            pl.BlockSpec((rb, d), lambda i: (i, 0)),
            pl.BlockSpec((rb, d // 2), lambda i: (i, 0)),
        ],
        out_shape=[
            jax.ShapeDtypeStruct((_N_PAD, d), jnp.float32),
            jax.ShapeDtypeStruct((_N_PAD, d // 2), jnp.int32),
        ],
    )(h, ha, wat, wbt_lo, wbt_hi, wpt, bp8)


def _edge_partition(src, dst):
    e_pad = src.shape[0]
    n_chunks = e_pad // _CHUNK
    n_pairs = (n_chunks + 1) // 2
    cap = ((e_pad + _SUP - 1) // _SUP) * _SUP
    mesh = plsc.VectorSubcoreMesh(core_axis_name="c", subcore_axis_name="s")

    @functools.partial(
        pl.kernel,
        mesh=mesh,
        compiler_params=_SC_PARAMS,
        out_type=[
            jax.ShapeDtypeStruct((_NW, cap), jnp.int32),
            jax.ShapeDtypeStruct((_NW, cap), jnp.int32),
            jax.ShapeDtypeStruct((_NW, _L), jnp.int32),
        ],
        scratch_types=[
            pltpu.VMEM((_CHUNK,), jnp.int32),
            pltpu.VMEM((_CHUNK,), jnp.int32),
            pltpu.VMEM((_CHUNK,), jnp.int32),
            pltpu.VMEM((_CHUNK,), jnp.int32),
            pltpu.VMEM((_PCAP,), jnp.int32),
            pltpu.VMEM((_PCAP,), jnp.int32),
            pltpu.SMEM((2,), jnp.int32),
            pltpu.VMEM((_L,), jnp.int32),
            pltpu.VMEM_SHARED((e_pad,), jnp.int32),
            pltpu.VMEM_SHARED((e_pad,), jnp.int32),
            pltpu.SemaphoreType.DMA,
            pltpu.SemaphoreType.DMA,
        ],
    )
    def part_kernel(src_hbm, dst_hbm, lsrc_hbm, ldst_hbm, cnt_hbm,
                    src_a, src_b, dst_a, dst_b, psrc, pdst, st, crow,
                    src_sh, dst_sh, sem0, sem1):
        cid = lax.axis_index("c")
        sid = lax.axis_index("s")
        wid = cid * 16 + sid
        lo = wid * _ROWS_W
        hi = lo + _ROWS_W

        izero = jnp.zeros((_L,), jnp.int32)

        @pl.loop(0, _PCAP, step=_L)
        def _(i):
            psrc[pl.ds(i, _L)] = izero

        st[0] = 0
        st[1] = 0

        @pl.when(sid == 0)
        def _():
            pltpu.sync_copy(src_hbm, src_sh)
            pltpu.sync_copy(dst_hbm, dst_sh)

        plsc.subcore_barrier()

        def stage(cix, sbuf, dbuf, sem):
            base = cix * _CHUNK
            pltpu.async_copy(src_sh.at[pl.ds(base, _CHUNK)], sbuf, sem)
            pltpu.async_copy(dst_sh.at[pl.ds(base, _CHUNK)], dbuf, sem)

        def wait(cix, sbuf, dbuf, sem):
            pltpu.make_async_copy(src_sh.at[pl.ds(cix * _CHUNK, _CHUNK)],
                                  sbuf, sem).wait()
            pltpu.make_async_copy(dst_sh.at[pl.ds(cix * _CHUNK, _CHUNK)],
                                  dbuf, sem).wait()

        def scan(sbuf, dbuf):
            kv = jnp.full((_L,), st[0] - 1, jnp.int32)

            def group(g, kv):
                i = g * _L
                dvec = dbuf[pl.ds(i, _L)]
                svec = sbuf[pl.ds(i, _L)]
                m = (dvec >= lo) & (dvec < hi)
                pos = kv + jnp.cumsum(m.astype(jnp.int32))
                plsc.store_scatter(psrc, [pos], svec, mask=m)
                plsc.store_scatter(pdst, [pos], dvec - lo, mask=m)
                return kv + plsc.all_reduce_population_count(m)

            kv = lax.fori_loop(0, _CHUNK // _L, group, kv, unroll=4)
            k = kv[0] + 1
            nb = k // _BLK

            def emit(b, carry):
                bo = st[1] + b
                pltpu.sync_copy(
                    psrc.at[pl.ds(b * _BLK, _BLK)],
                    lsrc_hbm.at[wid, pl.ds(bo * _BLK, _BLK)])
                pltpu.sync_copy(
                    pdst.at[pl.ds(b * _BLK, _BLK)],
                    ldst_hbm.at[wid, pl.ds(bo * _BLK, _BLK)])
                return carry

            lax.fori_loop(0, nb, emit, 0)
            st[1] = st[1] + nb
            base = nb * _BLK

            @pl.when(nb > 0)
            def _():
                for c in range(_BLK // _L):
                    off = c * _L
                    psrc[pl.ds(off, _L)] = psrc[pl.ds(base + off, _L)]
                    pdst[pl.ds(off, _L)] = pdst[pl.ds(base + off, _L)]

            st[0] = k - base

        stage(0, src_a, dst_a, sem0)
        stage(1, src_b, dst_b, sem1)

        @pl.loop(0, n_pairs)
        def _(t):
            c0 = 2 * t
            c1 = 2 * t + 1
            wait(c0, src_a, dst_a, sem0)
            scan(src_a, dst_a)

            @pl.when(c0 + 2 < n_chunks)
            def _():
                stage(c0 + 2, src_a, dst_a, sem0)

            @pl.when(c1 < n_chunks)
            def _():
                wait(c1, src_b, dst_b, sem1)
                scan(src_b, dst_b)

                @pl.when(c1 + 2 < n_chunks)
                def _():
                    stage(c1 + 2, src_b, dst_b, sem1)

        total = st[1] * _BLK + st[0]

        @pl.when(st[0] > 0)
        def _():
            b = st[1]
            pltpu.sync_copy(psrc.at[pl.ds(0, _BLK)],
                            lsrc_hbm.at[wid, pl.ds(b * _BLK, _BLK)])
            pltpu.sync_copy(pdst.at[pl.ds(0, _BLK)],
                            ldst_hbm.at[wid, pl.ds(b * _BLK, _BLK)])

        crow[...] = jnp.full((_L,), total, jnp.int32)
        pltpu.sync_copy(crow, cnt_hbm.at[wid])

    return part_kernel(src, dst)


def _fold(hp, lsrc, ldst, cnt):
    d = 2 * hp.shape[1]
    mesh = plsc.VectorSubcoreMesh(core_axis_name="c", subcore_axis_name="s")

    @functools.partial(
        pl.kernel,
        mesh=mesh,
        compiler_params=_SC_PARAMS,
        out_type=jax.ShapeDtypeStruct((_N_PAD, d // 2), jnp.int32),
        scratch_types=[
            pltpu.VMEM((_ROWS_W, d // 2), jnp.int32),
            pltpu.VMEM((_SUP,), jnp.int32),
            pltpu.VMEM((_SUP + 2 * _L,), jnp.int32),
            pltpu.VMEM((_BLK, d // 2), jnp.int32),
            pltpu.VMEM((_BLK, d // 2), jnp.int32),
            pltpu.VMEM((_BLK, d // 2), jnp.int32),
            pltpu.VMEM((_BLK, d // 2), jnp.int32),
            pltpu.VMEM((_L,), jnp.int32),
            pltpu.SemaphoreType.DMA,
            pltpu.SemaphoreType.DMA,
            pltpu.SemaphoreType.DMA,
            pltpu.SemaphoreType.DMA,
        ],
    )
    def fold_kernel(hp_hbm, lsrc_hbm, ldst_hbm, cnt_hbm, out_hbm,
                    acc, sidx, didx, rows_a, rows_b, rows_c, rows_d,
                    crow, sem0, sem1, sem2, sem3):
        cid = lax.axis_index("c")
        sid = lax.axis_index("s")
        wid = cid * 16 + sid
        lo = wid * _ROWS_W

        izero = jnp.zeros((_L,), jnp.int32)

        @pl.loop(0, _ROWS_W)
        def _(r):
            for c in range(d // (2 * _L)):
                acc[r, pl.ds(c * _L, _L)] = izero

        pltpu.sync_copy(cnt_hbm.at[wid], crow)
        total = crow[...][0]
        gmax = (total + _BLK - 1) // _BLK
        n_sup = (total + _SUP - 1) // _SUP

        def idx_ref(b):
            return sidx.at[pl.ds(b * _BLK, _BLK)]

        def issue(b, rows, sem):
            pltpu.async_copy(hp_hbm.at[idx_ref(b)], rows, sem)

        def fold_block(s, b, rows, sem, nbh):
            pltpu.make_async_copy(hp_hbm.at[idx_ref(b)], rows, sem).wait()
            gb = s * _BPS + b
            nb = jnp.minimum(total - gb * _BLK, _BLK)
            base = b * _BLK
            w = 2 * _L

            def fold_row(j, ld):
                ld_next = didx[pl.ds(base + j + 1, _L)][0]
                for c in range(0, d // w, 2):
                    sl0 = pl.ds(c * _L, _L)
                    sl1 = pl.ds((c + 1) * _L, _L)
                    a0 = plsc.bitcast(acc[ld, sl0], jnp.bfloat16)
                    r0 = plsc.bitcast(rows[j, sl0], jnp.bfloat16)
                    a1 = plsc.bitcast(acc[ld, sl1], jnp.bfloat16)
                    r1 = plsc.bitcast(rows[j, sl1], jnp.bfloat16)
                    acc[ld, sl0] = plsc.bitcast(jnp.maximum(a0, r0),
                                                jnp.int32)
                    acc[ld, sl1] = plsc.bitcast(jnp.maximum(a1, r1),
                                                jnp.int32)
                return ld_next

            ld0 = didx[pl.ds(base, _L)][0]

            @pl.when(nb == _BLK)
            def _():
                lax.fori_loop(0, _BLK, fold_row, ld0, unroll=2)

            @pl.when(nb < _BLK)
            def _():
                lax.fori_loop(0, nb, fold_row, ld0)

        @pl.loop(0, n_sup)
        def _(s):
            pltpu.sync_copy(lsrc_hbm.at[wid, pl.ds(s * _SUP, _SUP)], sidx)
            pltpu.sync_copy(ldst_hbm.at[wid, pl.ds(s * _SUP, _SUP)],
                            didx.at[pl.ds(0, _SUP)])
            nbh = jnp.minimum(gmax - s * _BPS, _BPS)
            bufs = ((rows_a, sem0), (rows_b, sem1), (rows_c, sem2),
                    (rows_d, sem3))

            for i, (rows, sem) in enumerate(bufs):
                @pl.when(nbh >= i + 1)
                def _(i=i, rows=rows, sem=sem):
                    issue(i, rows, sem)

            @pl.loop(0, (nbh + _NBUF - 1) // _NBUF)
            def _(t):
                for i, (rows, sem) in enumerate(bufs):
                    b = _NBUF * t + i

                    @pl.when(b < nbh)
                    def _(b=b, rows=rows, sem=sem):
                        fold_block(s, b, rows, sem, nbh)

                        @pl.when(b + _NBUF < nbh)
                        def _():
                            pltpu.async_copy(hp_hbm.at[idx_ref(b + _NBUF)],
                                             rows, sem)

        pltpu.sync_copy(acc, out_hbm.at[pl.ds(lo, _ROWS_W)])

    return fold_kernel(hp, lsrc, ldst, cnt)


def kernel(x, edge_index, Wp, bp, W1, W2):
    n, d = x.shape
    src = edge_index[0].astype(jnp.int32)
    dst = edge_index[1].astype(jnp.int32)
    e = src.shape[0]
    e_pad = ((e + _CHUNK - 1) // _CHUNK) * _CHUNK
    if e_pad != e:
        src = jnp.concatenate([src, jnp.zeros((e_pad - e,), jnp.int32)])
        dst = jnp.concatenate([dst, jnp.full((e_pad - e,), n, jnp.int32)])

    x_pad = jnp.zeros((_N_PAD, d), jnp.float32).at[:n].set(x)
    wpt = Wp.T
    bp8 = jnp.broadcast_to(bp.reshape(1, d), (8, d))
    w1t = W1.T
    w2t = W2.T
    k = d // 2
    wat1, wb1 = w1t[:d], w1t[d:]
    wat2, wb2 = w2t[:d], w2t[d:]
    wbl1, wbh1 = wb1[:k], wb1[k:]
    wbl2, wbh2 = wb2[:k], wb2[k:]

    lsrc, ldst, cnt = _edge_partition(src, dst)
    hp1 = _tc_pre(x_pad, wpt, bp8)
    ha1 = _fold(hp1, lsrc, ldst, cnt)
    h2, hp2 = _tc_post_hp(x_pad, ha1, wat1, wbl1, wbh1, wpt, bp8)
    ha2 = _fold(hp2, lsrc, ldst, cnt)
    h3 = _tc_post(h2, ha2, wat2, wbl2, wbh2)
    return h3[:n]


# speedup vs baseline: 5.4320x; 1.0756x over previous
import functools

import jax
import jax.numpy as jnp
from jax import lax
from jax.experimental import pallas as pl
from jax.experimental.pallas import tpu as pltpu
from jax.experimental.pallas import tpu_sc as plsc

_N_PAD = 10240
_NW = 32
_ROWS_W = _N_PAD // _NW
_CHUNK = 4000
_BLK = 128
_SUP = 4096
_BPS = _SUP // _BLK
_L = 16
_PCAP = _CHUNK + 2 * _BLK
_NBUF = 4

_SC_PARAMS = pltpu.CompilerParams(needs_layout_passes=False)


def _dot(a, b):
    return jnp.dot(a, b, preferred_element_type=jnp.float32,
                   precision=lax.Precision.DEFAULT)


def _pack_bf16(s):
    k = s.shape[1] // 2
    def bits(x):
        b = lax.bitcast_convert_type(x, jnp.int32)
        rnd = (lax.shift_right_logical(b, 16) & 1) + 0x7FFF
        return lax.shift_right_logical(b + rnd, 16)
    lo = bits(s[:, :k])
    hi = bits(s[:, k:])
    return lo | lax.shift_left(hi, 16)


def _unpack_bf16(w):
    lo = lax.bitcast_convert_type(lax.shift_left(w, 16), jnp.float32)
    hi = lax.bitcast_convert_type(w & jnp.int32(-65536), jnp.float32)
    return lo, hi


def _dot_fast(a, b):
    return jnp.dot(a.astype(jnp.bfloat16), b.astype(jnp.bfloat16),
                   preferred_element_type=jnp.float32)


def _pre_body(h_ref, wpt_ref, bp_ref, o_ref):
    o_ref[...] = _pack_bf16(
        jax.nn.sigmoid(_dot_fast(h_ref[...], wpt_ref[...]) + bp_ref[0:1, :]))


def _tc_pre(h, wpt, bp8):
    d = h.shape[1]
    rb = 1024
    return pl.pallas_call(
        _pre_body,
        grid=(_N_PAD // rb,),
        in_specs=[
            pl.BlockSpec((rb, d), lambda i: (i, 0)),
            pl.BlockSpec((d, d), lambda i: (0, 0)),
            pl.BlockSpec((8, d), lambda i: (0, 0)),
        ],
        out_specs=pl.BlockSpec((rb, d // 2), lambda i: (i, 0)),
        out_shape=jax.ShapeDtypeStruct((_N_PAD, d // 2), jnp.int32),
    )(h, wpt, bp8)


def _post_update(h, ha_words, wat, wbt_lo, wbt_hi):
    ha_lo, ha_hi = _unpack_bf16(ha_words)
    s = jax.nn.sigmoid(_dot(h, wat) + _dot(ha_lo, wbt_lo)
                       + _dot(ha_hi, wbt_hi))
    nrm = jnp.sqrt(jnp.sum(s * s, axis=1, keepdims=True))
    return s / jnp.maximum(nrm, 1e-12)


def _post_body(h_ref, ha_ref, wat_ref, wbl_ref, wbh_ref, o_ref):
    o_ref[...] = _post_update(h_ref[...], ha_ref[...], wat_ref[...],
                              wbl_ref[...], wbh_ref[...])


def _post_hp_body(h_ref, ha_ref, wat_ref, wbl_ref, wbh_ref, wpt_ref, bp_ref,
                  o_ref, ohp_ref):
    hn = _post_update(h_ref[...], ha_ref[...], wat_ref[...], wbl_ref[...],
                      wbh_ref[...])
    o_ref[...] = hn
    ohp_ref[...] = _pack_bf16(
        jax.nn.sigmoid(_dot_fast(hn, wpt_ref[...]) + bp_ref[0:1, :]))


def _tc_post(h, ha, wat, wbt_lo, wbt_hi):
    d = h.shape[1]
    rb = 1024
    return pl.pallas_call(
        _post_body,
        grid=(_N_PAD // rb,),
        in_specs=[
            pl.BlockSpec((rb, d), lambda i: (i, 0)),
            pl.BlockSpec((rb, d // 2), lambda i: (i, 0)),
            pl.BlockSpec((d, d), lambda i: (0, 0)),
            pl.BlockSpec((d // 2, d), lambda i: (0, 0)),
            pl.BlockSpec((d // 2, d), lambda i: (0, 0)),
        ],
        out_specs=pl.BlockSpec((rb, d), lambda i: (i, 0)),
        out_shape=jax.ShapeDtypeStruct((_N_PAD, d), jnp.float32),
    )(h, ha, wat, wbt_lo, wbt_hi)


def _tc_post_hp(h, ha, wat, wbt_lo, wbt_hi, wpt, bp8):
    d = h.shape[1]
    rb = 1024
    return pl.pallas_call(
        _post_hp_body,
        grid=(_N_PAD // rb,),
        in_specs=[
            pl.BlockSpec((rb, d), lambda i: (i, 0)),
            pl.BlockSpec((rb, d // 2), lambda i: (i, 0)),
            pl.BlockSpec((d, d), lambda i: (0, 0)),
            pl.BlockSpec((d // 2, d), lambda i: (0, 0)),
            pl.BlockSpec((d // 2, d), lambda i: (0, 0)),
            pl.BlockSpec((d, d), lambda i: (0, 0)),
            pl.BlockSpec((8, d), lambda i: (0, 0)),
        ],
        out_specs=[
            pl.BlockSpec((rb, d), lambda i: (i, 0)),
            pl.BlockSpec((rb, d // 2), lambda i: (i, 0)),
        ],
        out_shape=[
            jax.ShapeDtypeStruct((_N_PAD, d), jnp.float32),
            jax.ShapeDtypeStruct((_N_PAD, d // 2), jnp.int32),
        ],
    )(h, ha, wat, wbt_lo, wbt_hi, wpt, bp8)


def _edge_partition(src, dst):
    e_pad = src.shape[0]
    n_chunks = e_pad // _CHUNK
    n_pairs = (n_chunks + 1) // 2
    cap = ((e_pad + _SUP - 1) // _SUP) * _SUP
    mesh = plsc.VectorSubcoreMesh(core_axis_name="c", subcore_axis_name="s")

    @functools.partial(
        pl.kernel,
        mesh=mesh,
        compiler_params=_SC_PARAMS,
        out_type=[
            jax.ShapeDtypeStruct((_NW, cap), jnp.int32),
            jax.ShapeDtypeStruct((_NW, cap), jnp.int32),
            jax.ShapeDtypeStruct((_NW, _L), jnp.int32),
        ],
        scratch_types=[
            pltpu.VMEM((_CHUNK,), jnp.int32),
            pltpu.VMEM((_CHUNK,), jnp.int32),
            pltpu.VMEM((_CHUNK,), jnp.int32),
            pltpu.VMEM((_CHUNK,), jnp.int32),
            pltpu.VMEM((_PCAP,), jnp.int32),
            pltpu.VMEM((_PCAP,), jnp.int32),
            pltpu.SMEM((2,), jnp.int32),
            pltpu.VMEM((_L,), jnp.int32),
            pltpu.VMEM_SHARED((e_pad,), jnp.int32),
            pltpu.VMEM_SHARED((e_pad,), jnp.int32),
            pltpu.SemaphoreType.DMA,
            pltpu.SemaphoreType.DMA,
        ],
    )
    def part_kernel(src_hbm, dst_hbm, lsrc_hbm, ldst_hbm, cnt_hbm,
                    src_a, src_b, dst_a, dst_b, psrc, pdst, st, crow,
                    src_sh, dst_sh, sem0, sem1):
        cid = lax.axis_index("c")
        sid = lax.axis_index("s")
        wid = cid * 16 + sid
        lo = wid * _ROWS_W
        hi = lo + _ROWS_W

        izero = jnp.zeros((_L,), jnp.int32)

        @pl.loop(0, _PCAP, step=_L)
        def _(i):
            psrc[pl.ds(i, _L)] = izero

        st[0] = 0
        st[1] = 0

        @pl.when(sid == 0)
        def _():
            pltpu.sync_copy(src_hbm, src_sh)
            pltpu.sync_copy(dst_hbm, dst_sh)

        plsc.subcore_barrier()

        def stage(cix, sbuf, dbuf, sem):
            base = cix * _CHUNK
            pltpu.async_copy(src_sh.at[pl.ds(base, _CHUNK)], sbuf, sem)
            pltpu.async_copy(dst_sh.at[pl.ds(base, _CHUNK)], dbuf, sem)

        def wait(cix, sbuf, dbuf, sem):
            pltpu.make_async_copy(src_sh.at[pl.ds(cix * _CHUNK, _CHUNK)],
                                  sbuf, sem).wait()
            pltpu.make_async_copy(dst_sh.at[pl.ds(cix * _CHUNK, _CHUNK)],
                                  dbuf, sem).wait()

        def scan(sbuf, dbuf):
            kv = jnp.full((_L,), st[0] - 1, jnp.int32)

            def group(g, kv):
                i = g * _L
                dvec = dbuf[pl.ds(i, _L)]
                svec = sbuf[pl.ds(i, _L)]
                m = (dvec >= lo) & (dvec < hi)
                pos = kv + jnp.cumsum(m.astype(jnp.int32))
                plsc.store_scatter(psrc, [pos], svec, mask=m)
                plsc.store_scatter(pdst, [pos], dvec - lo, mask=m)
                return kv + plsc.all_reduce_population_count(m)

            kv = lax.fori_loop(0, _CHUNK // _L, group, kv, unroll=4)
            k = kv[0] + 1
            nb = k // _BLK

            def emit(b, carry):
                bo = st[1] + b
                pltpu.sync_copy(
                    psrc.at[pl.ds(b * _BLK, _BLK)],
                    lsrc_hbm.at[wid, pl.ds(bo * _BLK, _BLK)])
                pltpu.sync_copy(
                    pdst.at[pl.ds(b * _BLK, _BLK)],
                    ldst_hbm.at[wid, pl.ds(bo * _BLK, _BLK)])
                return carry

            lax.fori_loop(0, nb, emit, 0)
            st[1] = st[1] + nb
            base = nb * _BLK

            @pl.when(nb > 0)
            def _():
                for c in range(_BLK // _L):
                    off = c * _L
                    psrc[pl.ds(off, _L)] = psrc[pl.ds(base + off, _L)]
                    pdst[pl.ds(off, _L)] = pdst[pl.ds(base + off, _L)]

            st[0] = k - base

        stage(0, src_a, dst_a, sem0)
        stage(1, src_b, dst_b, sem1)

        @pl.loop(0, n_pairs)
        def _(t):
            c0 = 2 * t
            c1 = 2 * t + 1
            wait(c0, src_a, dst_a, sem0)
            scan(src_a, dst_a)

            @pl.when(c0 + 2 < n_chunks)
            def _():
                stage(c0 + 2, src_a, dst_a, sem0)

            @pl.when(c1 < n_chunks)
            def _():
                wait(c1, src_b, dst_b, sem1)
                scan(src_b, dst_b)

                @pl.when(c1 + 2 < n_chunks)
                def _():
                    stage(c1 + 2, src_b, dst_b, sem1)

        total = st[1] * _BLK + st[0]

        @pl.when(st[0] > 0)
        def _():
            b = st[1]
            pltpu.sync_copy(psrc.at[pl.ds(0, _BLK)],
                            lsrc_hbm.at[wid, pl.ds(b * _BLK, _BLK)])
            pltpu.sync_copy(pdst.at[pl.ds(0, _BLK)],
                            ldst_hbm.at[wid, pl.ds(b * _BLK, _BLK)])

        crow[...] = jnp.full((_L,), total, jnp.int32)
        pltpu.sync_copy(crow, cnt_hbm.at[wid])

    return part_kernel(src, dst)


def _fold(hp, lsrc, ldst, cnt):
    d = 2 * hp.shape[1]
    mesh = plsc.VectorSubcoreMesh(core_axis_name="c", subcore_axis_name="s")

    @functools.partial(
        pl.kernel,
        mesh=mesh,
        compiler_params=_SC_PARAMS,
        out_type=jax.ShapeDtypeStruct((_N_PAD, d // 2), jnp.int32),
        scratch_types=[
            pltpu.VMEM((_ROWS_W, d // 2), jnp.int32),
            pltpu.VMEM((_SUP,), jnp.int32),
            pltpu.VMEM((_SUP + 2 * _L,), jnp.int32),
            pltpu.VMEM((_BLK, d // 2), jnp.int32),
            pltpu.VMEM((_BLK, d // 2), jnp.int32),
            pltpu.VMEM((_BLK, d // 2), jnp.int32),
            pltpu.VMEM((_BLK, d // 2), jnp.int32),
            pltpu.VMEM((_L,), jnp.int32),
            pltpu.SemaphoreType.DMA,
            pltpu.SemaphoreType.DMA,
            pltpu.SemaphoreType.DMA,
            pltpu.SemaphoreType.DMA,
        ],
    )
    def fold_kernel(hp_hbm, lsrc_hbm, ldst_hbm, cnt_hbm, out_hbm,
                    acc, sidx, didx, rows_a, rows_b, rows_c, rows_d,
                    crow, sem0, sem1, sem2, sem3):
        cid = lax.axis_index("c")
        sid = lax.axis_index("s")
        wid = cid * 16 + sid
        lo = wid * _ROWS_W

        izero = jnp.zeros((_L,), jnp.int32)

        @pl.loop(0, _ROWS_W)
        def _(r):
            for c in range(d // (2 * _L)):
                acc[r, pl.ds(c * _L, _L)] = izero

        pltpu.sync_copy(cnt_hbm.at[wid], crow)
        total = crow[...][0]
        gmax = (total + _BLK - 1) // _BLK
        n_sup = (total + _SUP - 1) // _SUP

        def idx_ref(b):
            return sidx.at[pl.ds(b * _BLK, _BLK)]

        def issue(b, rows, sem):
            pltpu.async_copy(hp_hbm.at[idx_ref(b)], rows, sem)

        def fold_block(s, b, rows, sem, nbh):
            pltpu.make_async_copy(hp_hbm.at[idx_ref(b)], rows, sem).wait()
            gb = s * _BPS + b
            nb = jnp.minimum(total - gb * _BLK, _BLK)
            base = b * _BLK
            w = 2 * _L

            def fold_row(j, ld):
                ld_next = didx[pl.ds(base + j + 1, _L)][0]
                for c in range(0, d // w, 2):
                    sl0 = pl.ds(c * _L, _L)
                    sl1 = pl.ds((c + 1) * _L, _L)
                    a0 = plsc.bitcast(acc[ld, sl0], jnp.bfloat16)
                    r0 = plsc.bitcast(rows[j, sl0], jnp.bfloat16)
                    a1 = plsc.bitcast(acc[ld, sl1], jnp.bfloat16)
                    r1 = plsc.bitcast(rows[j, sl1], jnp.bfloat16)
                    acc[ld, sl0] = plsc.bitcast(jnp.maximum(a0, r0),
                                                jnp.int32)
                    acc[ld, sl1] = plsc.bitcast(jnp.maximum(a1, r1),
                                                jnp.int32)
                return ld_next

            ld0 = didx[pl.ds(base, _L)][0]

            @pl.when(nb == _BLK)
            def _():
                lax.fori_loop(0, _BLK, fold_row, ld0, unroll=2)

            @pl.when(nb < _BLK)
            def _():
                lax.fori_loop(0, nb, fold_row, ld0)

        @pl.loop(0, n_sup)
        def _(s):
            pltpu.sync_copy(lsrc_hbm.at[wid, pl.ds(s * _SUP, _SUP)], sidx)
            pltpu.sync_copy(ldst_hbm.at[wid, pl.ds(s * _SUP, _SUP)],
                            didx.at[pl.ds(0, _SUP)])
            nbh = jnp.minimum(gmax - s * _BPS, _BPS)
            bufs = ((rows_a, sem0), (rows_b, sem1), (rows_c, sem2),
                    (rows_d, sem3))

            for i, (rows, sem) in enumerate(bufs):
                @pl.when(nbh >= i + 1)
                def _(i=i, rows=rows, sem=sem):
                    issue(i, rows, sem)

            @pl.loop(0, (nbh + _NBUF - 1) // _NBUF)
            def _(t):
                for i, (rows, sem) in enumerate(bufs):
                    b = _NBUF * t + i

                    @pl.when(b < nbh)
                    def _(b=b, rows=rows, sem=sem):
                        fold_block(s, b, rows, sem, nbh)

                        @pl.when(b + _NBUF < nbh)
                        def _():
                            pltpu.async_copy(hp_hbm.at[idx_ref(b + _NBUF)],
                                             rows, sem)

        pltpu.sync_copy(acc, out_hbm.at[pl.ds(lo, _ROWS_W)])

    return fold_kernel(hp, lsrc, ldst, cnt)


def kernel(x, edge_index, Wp, bp, W1, W2):
    n, d = x.shape
    src = edge_index[0].astype(jnp.int32)
    dst = edge_index[1].astype(jnp.int32)
    e = src.shape[0]
    e_pad = ((e + _CHUNK - 1) // _CHUNK) * _CHUNK
    if e_pad != e:
        src = jnp.concatenate([src, jnp.zeros((e_pad - e,), jnp.int32)])
        dst = jnp.concatenate([dst, jnp.full((e_pad - e,), n, jnp.int32)])

    x_pad = jnp.zeros((_N_PAD, d), jnp.float32).at[:n].set(x)
    wpt = Wp.T
    bp8 = jnp.broadcast_to(bp.reshape(1, d), (8, d))
    w1t = W1.T
    w2t = W2.T
    k = d // 2
    wat1, wb1 = w1t[:d], w1t[d:]
    wat2, wb2 = w2t[:d], w2t[d:]
    wbl1, wbh1 = wb1[:k], wb1[k:]
    wbl2, wbh2 = wb2[:k], wb2[k:]

    lsrc, ldst, cnt = _edge_partition(src, dst)
    hp1 = _tc_pre(x_pad, wpt, bp8)
    ha1 = _fold(hp1, lsrc, ldst, cnt)
    h2, hp2 = _tc_post_hp(x_pad, ha1, wat1, wbl1, wbh1, wpt, bp8)
    ha2 = _fold(hp2, lsrc, ldst, cnt)
    h3 = _tc_post(h2, ha2, wat2, wbl2, wbh2)
    return h3[:n]
